# Initial kernel scaffold; baseline (speedup 1.0000x reference)
import jax
import jax.numpy as jnp
from jax.experimental import pallas as pl


def kernel(params, sm_x, sm_edge_index, sm_batch, p2_x, p2_edge_index, p2_batch, p3_x, p3_edge_index, p3_batch, solvent_a_x, solvent_a_edge_index, solvent_a_batch, solvent_b_x, solvent_b_edge_index, solvent_b_batch, percent_b, temperature, residence_time, drfp):
    raise NotImplementedError("write your pallas kernel here")


# speedup vs baseline: 1.0000x; 1.0000x over previous
import jax
import jax.numpy as jnp
from jax.experimental import pallas as pl

N = 10000
E = 320000
IN_DIM = 128
HID = 256
HEADS = 8
HD = HID // HEADS
LAYERS = 4
B = 256
DRFP = 2048
EDIM = 2 * HID


def _gat(h, src, dst, p, n):
    g = (h @ p["W"]).reshape(n, HEADS, HD)
    a_src = (g * p["att_src"][None]).sum(-1)
    a_dst = (g * p["att_dst"][None]).sum(-1)
    e = a_src[src] + a_dst[dst]
    e = jax.nn.leaky_relu(e, 0.2)
    emax = jax.ops.segment_max(e, dst, num_segments=n)
    emax = jnp.where(jnp.isfinite(emax), emax, 0.0)
    ex = jnp.exp(e - emax[dst])
    den = jax.ops.segment_sum(ex, dst, num_segments=n)
    alpha = ex / (den[dst] + 1e-16)
    out = jax.ops.segment_sum(g[src] * alpha[:, :, None], dst, num_segments=n)
    return out.reshape(n, HID) + p["bias"]


def _encode(p, x, edge_index, batch_idx):
    n = x.shape[0]
    loop = jnp.arange(n, dtype=edge_index.dtype)
    src = jnp.concatenate([edge_index[0], loop])
    dst = jnp.concatenate([edge_index[1], loop])
    h = x @ p["in_W"] + p["in_b"]
    for lp in p["layers"]:
        h = _gat(h, src, dst, lp, n) + h
    ones = jnp.ones((n,), jnp.float32)
    counts = jax.ops.segment_sum(ones, batch_idx, num_segments=B)
    mean = jax.ops.segment_sum(h, batch_idx, num_segments=B) / jnp.maximum(counts, 1.0)[:, None]
    mx = jax.ops.segment_max(h, batch_idx, num_segments=B)
    mx = jnp.where(counts[:, None] > 0, mx, 0.0)
    return jnp.concatenate([mean, mx], axis=-1)


def kernel(params, sm_x, sm_edge_index, sm_batch, p2_x, p2_edge_index, p2_batch, p3_x, p3_edge_index, p3_batch, solvent_a_x, solvent_a_edge_index, solvent_a_batch, solvent_b_x, solvent_b_edge_index, solvent_b_batch, percent_b, temperature, residence_time, drfp):
    e_sm = _encode(params["react"], sm_x, sm_edge_index, sm_batch)
    e_p2 = _encode(params["react"], p2_x, p2_edge_index, p2_batch)
    e_p3 = _encode(params["react"], p3_x, p3_edge_index, p3_batch)
    e_a = _encode(params["solv"], solvent_a_x, solvent_a_edge_index, solvent_a_batch)
    e_b = _encode(params["solv"], solvent_b_x, solvent_b_edge_index, solvent_b_batch)
    mix_in = jnp.concatenate([e_a, e_b, percent_b[:, None], temperature[:, None], residence_time[:, None]], axis=-1)
    e_mix = jax.nn.relu(mix_in @ params["mix_W1"] + params["mix_b1"]) @ params["mix_W2"] + params["mix_b2"]
    final_in = jnp.concatenate([e_sm, e_p2, e_p3, e_a, e_b, e_mix, drfp, temperature[:, None], residence_time[:, None], percent_b[:, None]], axis=-1)
    h = jax.nn.relu(final_in @ params["head_W1"] + params["head_b1"])
    return jax.nn.sigmoid(h @ params["head_W2"] + params["head_b2"])


# speedup vs baseline: 16.5050x; 16.5049x over previous
import functools

import jax
import jax.numpy as jnp
from jax import lax
from jax.experimental import pallas as pl
from jax.experimental.pallas import tpu as pltpu
from jax.experimental.pallas import tpu_sc as plsc

_N = 10000
_E = 320000
_IN = 128
_HID = 256
_HEADS = 8
_HD = 32
_B = 256
_NG = 5
_BN = 1000
_NBLK = _N // _BN
_C = 80
_EPW = _E // 16
_NCH = _EPW // _C
_RPS = _N // 16
_NEG = -1e30

_f32 = jnp.float32
_i32 = jnp.int32


def _sel(g):
    return jnp.where(g < 3, 0, 1)


def _in_body(x_ref, w_ref, b_ref, o_ref):
    o_ref[0] = jnp.dot(x_ref[0], w_ref[0], preferred_element_type=_f32) + b_ref[0]


_k_in = pl.pallas_call(
    _in_body,
    grid=(_NG, _NBLK),
    in_specs=[
        pl.BlockSpec((1, _BN, _IN), lambda g, i: (g, i, 0)),
        pl.BlockSpec((1, _IN, _HID), lambda g, i: (_sel(g), 0, 0)),
        pl.BlockSpec((1, 1, _HID), lambda g, i: (_sel(g), 0, 0)),
    ],
    out_specs=pl.BlockSpec((1, _BN, _HID), lambda g, i: (g, i, 0)),
    out_shape=jax.ShapeDtypeStruct((_NG, _N, _HID), _f32),
)


def _prep_body(h_ref, w_ref, vs_ref, vd_ref, g_ref, as_ref, ad_ref, ms_ref, md_ref):
    i = pl.program_id(1)
    g = jnp.dot(h_ref[0], w_ref[0], preferred_element_type=_f32)
    for q in range(8):
        g_ref[q, 0] = g[:, q * 32:(q + 1) * 32]
    a_s = jnp.dot(g, vs_ref[0], preferred_element_type=_f32)
    a_d = jnp.dot(g, vd_ref[0], preferred_element_type=_f32)
    as_ref[0, 0] = a_s[:, :4]
    as_ref[1, 0] = a_s[:, 4:]
    ad_ref[0, 0] = a_d[:, :4]
    ad_ref[1, 0] = a_d[:, 4:]
    pad = jnp.full((1, 120), _NEG, _f32)
    ms = jnp.concatenate([jnp.max(a_s, axis=0)[None, :], pad], axis=1)[None]
    md = jnp.concatenate([jnp.max(a_d, axis=0)[None, :], pad], axis=1)[None]

    @pl.when(i == 0)
    def _():
        ms_ref[...] = ms
        md_ref[...] = md

    @pl.when(i > 0)
    def _():
        ms_ref[...] = jnp.maximum(ms_ref[...], ms)
        md_ref[...] = jnp.maximum(md_ref[...], md)


_k_prep = pl.pallas_call(
    _prep_body,
    grid=(_NG, _NBLK),
    in_specs=[
        pl.BlockSpec((1, _BN, _HID), lambda g, i: (g, i, 0)),
        pl.BlockSpec((1, _HID, _HID), lambda g, i: (_sel(g), 0, 0)),
        pl.BlockSpec((1, _HID, _HEADS), lambda g, i: (_sel(g), 0, 0)),
        pl.BlockSpec((1, _HID, _HEADS), lambda g, i: (_sel(g), 0, 0)),
    ],
    out_specs=[
        pl.BlockSpec((8, 1, _BN, 32), lambda g, i: (0, g, i, 0)),
        pl.BlockSpec((2, 1, _BN, 4), lambda g, i: (0, g, i, 0)),
        pl.BlockSpec((2, 1, _BN, 4), lambda g, i: (0, g, i, 0)),
        pl.BlockSpec((1, 1, 128), lambda g, i: (g, 0, 0)),
        pl.BlockSpec((1, 1, 128), lambda g, i: (g, 0, 0)),
    ],
    out_shape=[
        jax.ShapeDtypeStruct((8, _NG, _N, 32), _f32),
        jax.ShapeDtypeStruct((2, _NG, _N, 4), _f32),
        jax.ShapeDtypeStruct((2, _NG, _N, 4), _f32),
        jax.ShapeDtypeStruct((_NG, 1, 128), _f32),
        jax.ShapeDtypeStruct((_NG, 1, 128), _f32),
    ],
    compiler_params=pltpu.CompilerParams(
        dimension_semantics=("arbitrary", "arbitrary")),
)


def _finish_body(num_ref, den_ref, as_ref, ad_ref, g_ref, ms_ref, md_ref,
                 b_ref, h_ref, o_ref):
    m = ms_ref[0, :, :8] + md_ref[0, :, :8]
    m = jnp.where(m >= 0.0, m, 0.2 * m)
    halves = []
    for c in (0, 1):
        a = as_ref[c, 0] + ad_ref[c, 0]
        el = jnp.where(a >= 0.0, a, 0.2 * a)
        ex = jnp.exp(el - m[:, c * 4:c * 4 + 4])
        den = den_ref[c, 0][:, :4] + ex
        gh = jnp.concatenate([g_ref[c * 4 + t, 0] for t in range(4)], axis=1)
        num = jnp.concatenate([num_ref[c * 4 + t, 0] for t in range(4)], axis=1)
        cols = []
        for hh in range(4):
            sl = slice(hh * 32, (hh + 1) * 32)
            numc = num[:, sl] + ex[:, hh:hh + 1] * gh[:, sl]
            cols.append(numc / (den[:, hh:hh + 1] + 1e-16))
        halves.append(jnp.concatenate(cols, axis=1))
    o_ref[0] = jnp.concatenate(halves, axis=1) + b_ref[0] + h_ref[0]


_k_finish = pl.pallas_call(
    _finish_body,
    grid=(_NG, _NBLK),
    in_specs=[
        pl.BlockSpec((8, 1, _BN, 32), lambda g, i: (0, g, i, 0)),
        pl.BlockSpec((2, 1, _BN, 8), lambda g, i: (0, g, i, 0)),
        pl.BlockSpec((2, 1, _BN, 4), lambda g, i: (0, g, i, 0)),
        pl.BlockSpec((2, 1, _BN, 4), lambda g, i: (0, g, i, 0)),
        pl.BlockSpec((8, 1, _BN, 32), lambda g, i: (0, g, i, 0)),
        pl.BlockSpec((1, 1, 128), lambda g, i: (g, 0, 0)),
        pl.BlockSpec((1, 1, 128), lambda g, i: (g, 0, 0)),
        pl.BlockSpec((1, 1, _HID), lambda g, i: (_sel(g), 0, 0)),
        pl.BlockSpec((1, _BN, _HID), lambda g, i: (g, i, 0)),
    ],
    out_specs=pl.BlockSpec((1, _BN, _HID), lambda g, i: (g, i, 0)),
    out_shape=jax.ShapeDtypeStruct((_NG, _N, _HID), _f32),
)


_mesh = plsc.VectorSubcoreMesh(
    core_axis_name="c", subcore_axis_name="s", num_cores=2, num_subcores=16)


@functools.partial(
    pl.kernel,
    out_type=[
        jax.ShapeDtypeStruct((8, _NG, _N, 32), _f32),
        jax.ShapeDtypeStruct((2, _NG, _N, 8), _f32),
    ],
    mesh=_mesh,
    scratch_types=[
        pltpu.VMEM((_N * 4,), _f32),
        pltpu.VMEM((_N * 4,), _f32),
        pltpu.VMEM((_C,), _i32),
        pltpu.VMEM((_C,), _i32),
        pltpu.VMEM((_C, 32), _f32),
        pltpu.VMEM((_C, 8), _f32),
        pltpu.VMEM((125, 32), _f32),
        pltpu.VMEM((128, 8), _f32),
        pltpu.VMEM((16,), _f32),
        pltpu.VMEM((16,), _f32),
        pltpu.VMEM((64,), _f32),
        pltpu.VMEM_SHARED((_N, 32), _f32),
        pltpu.VMEM_SHARED((_N, 8), _f32),
        pltpu.SemaphoreType.DMA,
    ],
    compiler_params=pltpu.CompilerParams(use_tc_tiling_on_sc=False, needs_layout_passes=False),
)
def _k_edge(src_hbm, dst_hbm, as_hbm, ad_hbm, g_hbm, ms_hbm, md_hbm,
            num_out, den_out, asrc_t, adst_t, src_b, dst_b, rows, exb,
            zb, dzb, mt1, mt2, mloc, num_sp, den_sp, sem):
    c = lax.axis_index("c")
    s = lax.axis_index("s")
    zero16 = jnp.zeros((16,), _f32)
    iota = lax.iota(_i32, 16)

    def _zb(i, carry):
        zb[i // 2, pl.ds((i % 2) * 16, 16)] = zero16
        return carry

    lax.fori_loop(0, 250, _zb, 0)

    def _dzb(j, carry):
        for cc in range(8):
            plsc.store_scatter(
                dzb, [j * 16 + iota, jnp.full((16,), cc, _i32)], zero16)
        return carry

    lax.fori_loop(0, 8, _dzb, 0)

    for g in range(_NG):
        pltpu.sync_copy(as_hbm.at[c, g], asrc_t)
        pltpu.sync_copy(ad_hbm.at[c, g], adst_t)
        pltpu.sync_copy(ms_hbm.at[g, 0, pl.ds(0, 16)], mt1)
        pltpu.sync_copy(md_hbm.at[g, 0, pl.ds(0, 16)], mt2)
        mv = mt1[:] + mt2[:]
        mt2[:] = jnp.where(mv >= 0.0, mv, 0.2 * mv)
        for hh in range(4):
            hidx = jnp.zeros((16,), _i32) + (c * 4 + hh)
            mloc[pl.ds(hh * 16, 16)] = plsc.load_gather(mt2, [hidx])

        for p in range(4):
            for t in range(5):
                r0 = s * _RPS + t * 125
                pltpu.sync_copy(zb, num_sp.at[pl.ds(r0, 125), :])
                if p == 0:
                    pltpu.sync_copy(dzb.at[pl.ds(0, 125), :],
                                    den_sp.at[pl.ds(r0, 125), :])

            def _exz(j, carry):
                for cc in range(8):
                    plsc.store_scatter(
                        exb, [j * 16 + iota, jnp.full((16,), cc, _i32)], zero16)
                return carry

            lax.fori_loop(0, _C // 16, _exz, 0)
            plsc.subcore_barrier()

            def _chunk(ch, carry):
                base = s * _EPW + ch * _C
                pltpu.sync_copy(src_hbm.at[g, pl.ds(base, _C)], src_b)
                pltpu.sync_copy(dst_hbm.at[g, pl.ds(base, _C)], dst_b)
                pltpu.async_copy(
                    g_hbm.at[c * 4 + p, g].at[src_b], rows, sem).wait()

                def _group(j, carry2):
                    src16 = src_b[pl.ds(j * 16, 16)]
                    dst16 = dst_b[pl.ds(j * 16, 16)]
                    e = (plsc.load_gather(asrc_t, [src16 * 4 + p])
                         + plsc.load_gather(adst_t, [dst16 * 4 + p]))
                    e = jnp.where(e >= 0.0, e, 0.2 * e)
                    ex = jnp.exp(e - mloc[pl.ds(p * 16, 16)])
                    plsc.store_scatter(
                        exb, [j * 16 + iota, jnp.full((16,), p, _i32)], ex)
                    return carry2

                lax.fori_loop(0, _C // 16, _group, 0)

                def _scale(ed, carry2):
                    exv = plsc.load_gather(
                        exb, [jnp.zeros((16,), _i32) + ed,
                              jnp.full((16,), p, _i32)])
                    for q in range(2):
                        rows[ed, pl.ds(q * 16, 16)] = (
                            rows[ed, pl.ds(q * 16, 16)] * exv)
                    return carry2

                lax.fori_loop(0, _C, _scale, 0)
                pltpu.sync_copy(exb, den_sp.at[dst_b], add=True)
                pltpu.sync_copy(rows, num_sp.at[dst_b], add=True)
                return carry

            lax.fori_loop(0, _NCH, _chunk, 0)
            plsc.subcore_barrier()
            r0 = s * _RPS
            pltpu.sync_copy(num_sp.at[pl.ds(r0, _RPS), :],
                            num_out.at[c * 4 + p, g, pl.ds(r0, _RPS), :])
            if p == 3:
                pltpu.sync_copy(den_sp.at[pl.ds(r0, _RPS), :],
                                den_out.at[c, g, pl.ds(r0, _RPS), :])


@functools.partial(
    pl.kernel,
    out_type=[
        jax.ShapeDtypeStruct((_NG, _B, _HID), _f32),
        jax.ShapeDtypeStruct((_NG, _B, _HID), _f32),
    ],
    mesh=_mesh,
    scratch_types=[
        pltpu.VMEM((_N,), _i32),
        pltpu.VMEM((64, _HID), _f32),
        pltpu.VMEM((8, _HID), _f32),
        pltpu.VMEM((8, _HID), _f32),
        pltpu.VMEM((8, 16), _f32),
        pltpu.SemaphoreType.DMA,
    ],
    compiler_params=pltpu.CompilerParams(use_tc_tiling_on_sc=False, needs_layout_passes=False),
)
def _k_readout(h_hbm, batch_hbm, mean_out, max_out,
               bbuf, chunk, msum, mmax, cnt, sem):
    c = lax.axis_index("c")
    s = lax.axis_index("s")
    w = s * 2 + c
    lo = w * 8
    zero16 = jnp.zeros((16,), _f32)
    lov = jnp.zeros((16,), _i32) + lo
    hiv = lov + 8

    def _per_graph(g, carry):
        pltpu.sync_copy(batch_hbm.at[g], bbuf)

        def _cnt(i, cr):
            n0, n1 = cr
            bv = bbuf[pl.ds(i * 16, 16)]
            n0 = n0 + plsc.all_reduce_population_count(bv < lov)
            n1 = n1 + plsc.all_reduce_population_count(bv < hiv)
            return (n0, n1)

        n0v, n1v = lax.fori_loop(
            0, _N // 16, _cnt,
            (jnp.zeros((16,), _i32), jnp.zeros((16,), _i32)))
        n0 = jnp.max(n0v)
        nn = jnp.max(n1v) - n0

        def _init(r, cr):
            for q in range(16):
                msum[r, pl.ds(q * 16, 16)] = zero16
                mmax[r, pl.ds(q * 16, 16)] = zero16 + _NEG
            cnt[r, :] = zero16
            return cr

        lax.fori_loop(0, 8, _init, 0)

        nch = (nn + 63) // 64

        def _chunkb(k, cr):
            start = n0 + k * 64
            r0 = jnp.minimum(start, _N - 64)
            pltpu.sync_copy(h_hbm.at[g, pl.ds(r0, 64), :], chunk)
            startv = jnp.zeros((16,), _i32) + start

            def _row(r, cr2):
                node = r0 + r
                nodev = jnp.zeros((16,), _i32) + node
                bn16 = plsc.load_gather(bbuf, [nodev])
                valid = (nodev >= startv) & (nodev < (jnp.zeros((16,), _i32) + n0 + nn))
                local = jnp.clip(bn16 - lov, 0, 7)
                lsc = jnp.max(local)
                for q in range(16):
                    v = chunk[r, pl.ds(q * 16, 16)]
                    msum[lsc, pl.ds(q * 16, 16)] = (
                        msum[lsc, pl.ds(q * 16, 16)] + jnp.where(valid, v, 0.0))
                    mmax[lsc, pl.ds(q * 16, 16)] = jnp.maximum(
                        mmax[lsc, pl.ds(q * 16, 16)], jnp.where(valid, v, _NEG))
                cnt[lsc, :] = cnt[lsc, :] + jnp.where(valid, 1.0, 0.0)
                return cr2

            lax.fori_loop(0, 64, _row, 0)
            return cr

        lax.fori_loop(0, nch, _chunkb, 0)

        def _fin(r, cr):
            cv = cnt[r, :]
            denom = jnp.maximum(cv, 1.0)
            empty = cv <= 0.0
            for q in range(16):
                msum[r, pl.ds(q * 16, 16)] = msum[r, pl.ds(q * 16, 16)] / denom
                mx = mmax[r, pl.ds(q * 16, 16)]
                mmax[r, pl.ds(q * 16, 16)] = jnp.where(empty, 0.0, mx)
            return cr

        lax.fori_loop(0, 8, _fin, 0)
        pltpu.sync_copy(msum, mean_out.at[g, pl.ds(lo, 8), :])
        pltpu.sync_copy(mmax, max_out.at[g, pl.ds(lo, 8), :])
        return carry

    lax.fori_loop(0, _NG, _per_graph, 0)


def _head_body(mean_ref, max_ref, drfp_ref, scm_ref, sch_ref,
               mW1_ref, mb1_ref, mW2_ref, mb2_ref,
               hW1_ref, hb1_ref, hW2_ref, hb2_ref, o_ref):
    def dot(a, b):
        return jnp.dot(a, b, preferred_element_type=_f32)

    acc1 = (dot(mean_ref[3], mW1_ref[0:256])
            + dot(max_ref[3], mW1_ref[256:512])
            + dot(mean_ref[4], mW1_ref[512:768])
            + dot(max_ref[4], mW1_ref[768:1024])
            + dot(scm_ref[...], mW1_ref[1024:1027])
            + mb1_ref[...])
    e_mix = dot(jnp.maximum(acc1, 0.0), mW2_ref[...]) + mb2_ref[...]
    acc = hb1_ref[...] + dot(e_mix, hW1_ref[2560:3072])
    for i in range(5):
        acc = acc + dot(mean_ref[i], hW1_ref[i * 512:i * 512 + 256])
        acc = acc + dot(max_ref[i], hW1_ref[i * 512 + 256:(i + 1) * 512])
    acc = acc + dot(drfp_ref[...], hW1_ref[3072:5120])
    acc = acc + dot(sch_ref[...], hW1_ref[5120:5123])
    h = jnp.maximum(acc, 0.0)
    z = dot(h, hW2_ref[...]) + hb2_ref[...]
    o_ref[...] = 1.0 / (1.0 + jnp.exp(-z))


_k_head = pl.pallas_call(
    _head_body,
    out_shape=jax.ShapeDtypeStruct((_B, 3), _f32),
)


def _att_mat(att):
    eye = jnp.eye(_HEADS, dtype=_f32)
    return (att[:, :, None] * eye[:, None, :]).reshape(_HID, _HEADS)


def kernel(params, sm_x, sm_edge_index, sm_batch, p2_x, p2_edge_index,
           p2_batch, p3_x, p3_edge_index, p3_batch, solvent_a_x,
           solvent_a_edge_index, solvent_a_batch, solvent_b_x,
           solvent_b_edge_index, solvent_b_batch, percent_b, temperature,
           residence_time, drfp):
    pr, ps = params["react"], params["solv"]
    x_all = jnp.stack([sm_x, p2_x, p3_x, solvent_a_x, solvent_b_x])
    eis = [sm_edge_index, p2_edge_index, p3_edge_index,
           solvent_a_edge_index, solvent_b_edge_index]
    src_all = jnp.stack([e[0] for e in eis])
    dst_all = jnp.stack([e[1] for e in eis])
    batch_all = jnp.stack([sm_batch, p2_batch, p3_batch,
                           solvent_a_batch, solvent_b_batch])

    w_in = jnp.stack([pr["in_W"], ps["in_W"]])
    b_in = jnp.stack([pr["in_b"], ps["in_b"]])[:, None, :]
    h = _k_in(x_all, w_in, b_in)

    for l in range(4):
        lr, ls = pr["layers"][l], ps["layers"][l]
        w = jnp.stack([lr["W"], ls["W"]])
        vs = jnp.stack([_att_mat(lr["att_src"]), _att_mat(ls["att_src"])])
        vd = jnp.stack([_att_mat(lr["att_dst"]), _att_mat(ls["att_dst"])])
        bias = jnp.stack([lr["bias"], ls["bias"]])[:, None, :]
        g_st, a_s, a_d, ms, md = _k_prep(h, w, vs, vd)
        num_st, den_st = _k_edge(src_all, dst_all,
                                 a_s.reshape(2, _NG, _N * 4),
                                 a_d.reshape(2, _NG, _N * 4), g_st, ms, md)
        h = _k_finish(num_st, den_st, a_s, a_d, g_st, ms, md, bias, h)

    mean_all, max_all = _k_readout(h, batch_all)

    scm = jnp.stack([percent_b, temperature, residence_time], axis=1)
    sch = jnp.stack([temperature, residence_time, percent_b], axis=1)
    return _k_head(mean_all, max_all, drfp, scm, sch,
                   params["mix_W1"], params["mix_b1"][None, :],
                   params["mix_W2"], params["mix_b2"][None, :],
                   params["head_W1"], params["head_b1"][None, :],
                   params["head_W2"], params["head_b2"][None, :])


# speedup vs baseline: 27.3753x; 1.6586x over previous
import functools

import jax
import jax.numpy as jnp
from jax import lax
from jax.experimental import pallas as pl
from jax.experimental.pallas import tpu as pltpu
from jax.experimental.pallas import tpu_sc as plsc

_N = 10000
_E = 320000
_IN = 128
_HID = 256
_HEADS = 8
_HD = 32
_B = 256
_NG = 5
_BN = 1000
_NBLK = _N // _BN
_C = 80
_EPW = _E // 16
_NCH = _EPW // _C
_RPS = _N // 16
_NEG = -1e30

_f32 = jnp.float32
_i32 = jnp.int32


def _sel(g):
    return jnp.where(g < 3, 0, 1)


def _in_body(x_ref, w_ref, b_ref, o_ref):
    o_ref[0] = jnp.dot(x_ref[0], w_ref[0], preferred_element_type=_f32) + b_ref[0]


_k_in = pl.pallas_call(
    _in_body,
    grid=(_NG, _NBLK),
    in_specs=[
        pl.BlockSpec((1, _BN, _IN), lambda g, i: (g, i, 0)),
        pl.BlockSpec((1, _IN, _HID), lambda g, i: (_sel(g), 0, 0)),
        pl.BlockSpec((1, 1, _HID), lambda g, i: (_sel(g), 0, 0)),
    ],
    out_specs=pl.BlockSpec((1, _BN, _HID), lambda g, i: (g, i, 0)),
    out_shape=jax.ShapeDtypeStruct((_NG, _N, _HID), _f32),
)


def _prep_body(h_ref, w_ref, vs_ref, vd_ref, g_ref, as_ref, ad_ref, ms_ref, md_ref):
    i = pl.program_id(1)
    g = jnp.dot(h_ref[0], w_ref[0], preferred_element_type=_f32)
    for q in range(8):
        g_ref[q, 0] = g[:, q * 32:(q + 1) * 32]
    a_s = jnp.dot(g, vs_ref[0], preferred_element_type=_f32)
    a_d = jnp.dot(g, vd_ref[0], preferred_element_type=_f32)
    as_ref[0, 0] = a_s[:, :4]
    as_ref[1, 0] = a_s[:, 4:]
    ad_ref[0, 0] = a_d[:, :4]
    ad_ref[1, 0] = a_d[:, 4:]
    pad = jnp.full((1, 120), _NEG, _f32)
    ms = jnp.concatenate([jnp.max(a_s, axis=0)[None, :], pad], axis=1)[None]
    md = jnp.concatenate([jnp.max(a_d, axis=0)[None, :], pad], axis=1)[None]

    @pl.when(i == 0)
    def _():
        ms_ref[...] = ms
        md_ref[...] = md

    @pl.when(i > 0)
    def _():
        ms_ref[...] = jnp.maximum(ms_ref[...], ms)
        md_ref[...] = jnp.maximum(md_ref[...], md)


_k_prep = pl.pallas_call(
    _prep_body,
    grid=(_NG, _NBLK),
    in_specs=[
        pl.BlockSpec((1, _BN, _HID), lambda g, i: (g, i, 0)),
        pl.BlockSpec((1, _HID, _HID), lambda g, i: (_sel(g), 0, 0)),
        pl.BlockSpec((1, _HID, _HEADS), lambda g, i: (_sel(g), 0, 0)),
        pl.BlockSpec((1, _HID, _HEADS), lambda g, i: (_sel(g), 0, 0)),
    ],
    out_specs=[
        pl.BlockSpec((8, 1, _BN, 32), lambda g, i: (0, g, i, 0)),
        pl.BlockSpec((2, 1, _BN, 4), lambda g, i: (0, g, i, 0)),
        pl.BlockSpec((2, 1, _BN, 4), lambda g, i: (0, g, i, 0)),
        pl.BlockSpec((1, 1, 128), lambda g, i: (g, 0, 0)),
        pl.BlockSpec((1, 1, 128), lambda g, i: (g, 0, 0)),
    ],
    out_shape=[
        jax.ShapeDtypeStruct((8, _NG, _N, 32), _f32),
        jax.ShapeDtypeStruct((2, _NG, _N, 4), _f32),
        jax.ShapeDtypeStruct((2, _NG, _N, 4), _f32),
        jax.ShapeDtypeStruct((_NG, 1, 128), _f32),
        jax.ShapeDtypeStruct((_NG, 1, 128), _f32),
    ],
    compiler_params=pltpu.CompilerParams(
        dimension_semantics=("arbitrary", "arbitrary")),
)


def _finish_body(num_ref, den_ref, as_ref, ad_ref, g_ref, ms_ref, md_ref,
                 b_ref, h_ref, o_ref):
    m = ms_ref[0, :, :8] + md_ref[0, :, :8]
    m = jnp.where(m >= 0.0, m, 0.2 * m)
    halves = []
    for c in (0, 1):
        a = as_ref[c, 0] + ad_ref[c, 0]
        el = jnp.where(a >= 0.0, a, 0.2 * a)
        ex = jnp.exp(el - m[:, c * 4:c * 4 + 4])
        den = den_ref[c, 0][:, :4] + ex
        gh = jnp.concatenate([g_ref[c * 4 + t, 0] for t in range(4)], axis=1)
        num = jnp.concatenate([num_ref[c * 4 + t, 0] for t in range(4)], axis=1)
        cols = []
        for hh in range(4):
            sl = slice(hh * 32, (hh + 1) * 32)
            numc = num[:, sl] + ex[:, hh:hh + 1] * gh[:, sl]
            cols.append(numc / (den[:, hh:hh + 1] + 1e-16))
        halves.append(jnp.concatenate(cols, axis=1))
    o_ref[0] = jnp.concatenate(halves, axis=1) + b_ref[0] + h_ref[0]


_k_finish = pl.pallas_call(
    _finish_body,
    grid=(_NG, _NBLK),
    in_specs=[
        pl.BlockSpec((8, 1, _BN, 32), lambda g, i: (0, g, i, 0)),
        pl.BlockSpec((2, 1, _BN, 8), lambda g, i: (0, g, i, 0)),
        pl.BlockSpec((2, 1, _BN, 4), lambda g, i: (0, g, i, 0)),
        pl.BlockSpec((2, 1, _BN, 4), lambda g, i: (0, g, i, 0)),
        pl.BlockSpec((8, 1, _BN, 32), lambda g, i: (0, g, i, 0)),
        pl.BlockSpec((1, 1, 128), lambda g, i: (g, 0, 0)),
        pl.BlockSpec((1, 1, 128), lambda g, i: (g, 0, 0)),
        pl.BlockSpec((1, 1, _HID), lambda g, i: (_sel(g), 0, 0)),
        pl.BlockSpec((1, _BN, _HID), lambda g, i: (g, i, 0)),
    ],
    out_specs=pl.BlockSpec((1, _BN, _HID), lambda g, i: (g, i, 0)),
    out_shape=jax.ShapeDtypeStruct((_NG, _N, _HID), _f32),
)


_mesh = plsc.VectorSubcoreMesh(
    core_axis_name="c", subcore_axis_name="s", num_cores=2, num_subcores=16)


@functools.partial(
    pl.kernel,
    out_type=[
        jax.ShapeDtypeStruct((8, _NG, _N, 32), _f32),
        jax.ShapeDtypeStruct((2, _NG, _N, 8), _f32),
    ],
    mesh=_mesh,
    scratch_types=[
        pltpu.VMEM((_N * 4,), _f32),
        pltpu.VMEM((_N * 4,), _f32),
        pltpu.VMEM((2, _C), _i32),
        pltpu.VMEM((2, _C), _i32),
        pltpu.VMEM((2, _C, 32), _f32),
        pltpu.VMEM((2, _C, 8), _f32),
        pltpu.VMEM((125, 32), _f32),
        pltpu.VMEM((128, 8), _f32),
        pltpu.VMEM((16,), _f32),
        pltpu.VMEM((16,), _f32),
        pltpu.VMEM((64,), _f32),
        pltpu.VMEM_SHARED((_N, 32), _f32),
        pltpu.VMEM_SHARED((_N, 8), _f32),
        pltpu.SemaphoreType.DMA,
        pltpu.SemaphoreType.DMA,
        pltpu.SemaphoreType.DMA,
        pltpu.SemaphoreType.DMA,
        pltpu.SemaphoreType.DMA,
        pltpu.SemaphoreType.DMA,
        pltpu.SemaphoreType.DMA,
        pltpu.SemaphoreType.DMA,
    ],
    compiler_params=pltpu.CompilerParams(use_tc_tiling_on_sc=False, needs_layout_passes=False),
)
def _k_edge(src_hbm, dst_hbm, as_hbm, ad_hbm, g_hbm, ms_hbm, md_hbm,
            num_out, den_out, asrc_t, adst_t, src_b, dst_b, rows, exb,
            zb, dzb, mt1, mt2, mloc, num_sp, den_sp,
            sIs0, sId0, sIs1, sId1, sG0, sG1, sSd, sSn):
    sIs = [sIs0, sIs1]
    sId = [sId0, sId1]
    sG = [sG0, sG1]
    c = lax.axis_index("c")
    s = lax.axis_index("s")
    zero16 = jnp.zeros((16,), _f32)
    iota = lax.iota(_i32, 16)

    def _zb(i, carry):
        zb[i // 2, pl.ds((i % 2) * 16, 16)] = zero16
        return carry

    lax.fori_loop(0, 250, _zb, 0)

    def _dzb(j, carry):
        for cc in range(8):
            plsc.store_scatter(
                dzb, [j * 16 + iota, jnp.full((16,), cc, _i32)], zero16)
        return carry

    lax.fori_loop(0, 8, _dzb, 0)

    for g in range(_NG):
        pltpu.sync_copy(as_hbm.at[c, g], asrc_t)
        pltpu.sync_copy(ad_hbm.at[c, g], adst_t)
        pltpu.sync_copy(ms_hbm.at[g, 0, pl.ds(0, 16)], mt1)
        pltpu.sync_copy(md_hbm.at[g, 0, pl.ds(0, 16)], mt2)
        mv = mt1[:] + mt2[:]
        mt2[:] = jnp.where(mv >= 0.0, mv, 0.2 * mv)
        for hh in range(4):
            hidx = jnp.zeros((16,), _i32) + (c * 4 + hh)
            mloc[pl.ds(hh * 16, 16)] = plsc.load_gather(mt2, [hidx])

        for p in range(4):
            for t in range(5):
                r0 = s * _RPS + t * 125
                pltpu.sync_copy(zb, num_sp.at[pl.ds(r0, 125), :])
                if p == 0:
                    pltpu.sync_copy(dzb.at[pl.ds(0, 125), :],
                                    den_sp.at[pl.ds(r0, 125), :])

            def _exz(j, carry):
                for bb in range(2):
                    for cc in range(8):
                        plsc.store_scatter(
                            exb.at[bb],
                            [j * 16 + iota, jnp.full((16,), cc, _i32)], zero16)
                return carry

            lax.fori_loop(0, _C // 16, _exz, 0)
            plsc.subcore_barrier()

            base0 = s * _EPW
            for bb in range(2):
                pltpu.async_copy(src_hbm.at[g, pl.ds(base0 + bb * _C, _C)],
                                 src_b.at[bb], sIs[bb])
                pltpu.async_copy(dst_hbm.at[g, pl.ds(base0 + bb * _C, _C)],
                                 dst_b.at[bb], sId[bb])

            def _pair(k2, carry):
                for bb in range(2):
                    ch = k2 * 2 + bb
                    base = s * _EPW + ch * _C
                    pltpu.make_async_copy(
                        src_hbm.at[g, pl.ds(base, _C)],
                        src_b.at[bb], sIs[bb]).wait()
                    pltpu.make_async_copy(
                        dst_hbm.at[g, pl.ds(base, _C)],
                        dst_b.at[bb], sId[bb]).wait()
                    gd = pltpu.async_copy(
                        g_hbm.at[c * 4 + p, g].at[src_b.at[bb]],
                        rows.at[bb], sG[bb])

                    def _group(j, carry2):
                        src16 = src_b[bb, pl.ds(j * 16, 16)]
                        dst16 = dst_b[bb, pl.ds(j * 16, 16)]
                        e = (plsc.load_gather(asrc_t, [src16 * 4 + p])
                             + plsc.load_gather(adst_t, [dst16 * 4 + p]))
                        e = jnp.where(e >= 0.0, e, 0.2 * e)
                        ex = jnp.exp(e - mloc[pl.ds(p * 16, 16)])
                        plsc.store_scatter(
                            exb.at[bb],
                            [j * 16 + iota, jnp.full((16,), p, _i32)], ex)
                        return carry2

                    lax.fori_loop(0, _C // 16, _group, 0)
                    gd.wait()

                    def _scale(ed, carry2):
                        exv = plsc.load_gather(
                            exb.at[bb], [jnp.zeros((16,), _i32) + ed,
                                         jnp.full((16,), p, _i32)])
                        for q in range(2):
                            rows[bb, ed, pl.ds(q * 16, 16)] = (
                                rows[bb, ed, pl.ds(q * 16, 16)] * exv)
                        return carry2

                    lax.fori_loop(0, _C, _scale, 0)
                    d1 = pltpu.async_copy(
                        exb.at[bb], den_sp.at[dst_b.at[bb]], sSd, add=True)
                    d2 = pltpu.async_copy(
                        rows.at[bb], num_sp.at[dst_b.at[bb]], sSn, add=True)
                    d1.wait()
                    d2.wait()

                    @pl.when(ch + 2 < _NCH)
                    def _():
                        pltpu.async_copy(
                            src_hbm.at[g, pl.ds(base + 2 * _C, _C)],
                            src_b.at[bb], sIs[bb])
                        pltpu.async_copy(
                            dst_hbm.at[g, pl.ds(base + 2 * _C, _C)],
                            dst_b.at[bb], sId[bb])
                return carry

            lax.fori_loop(0, _NCH // 2, _pair, 0)
            plsc.subcore_barrier()
            r0 = s * _RPS
            pltpu.sync_copy(num_sp.at[pl.ds(r0, _RPS), :],
                            num_out.at[c * 4 + p, g, pl.ds(r0, _RPS), :])
            if p == 3:
                pltpu.sync_copy(den_sp.at[pl.ds(r0, _RPS), :],
                                den_out.at[c, g, pl.ds(r0, _RPS), :])


@functools.partial(
    pl.kernel,
    out_type=[
        jax.ShapeDtypeStruct((_NG, _B, _HID), _f32),
        jax.ShapeDtypeStruct((_NG, _B, _HID), _f32),
    ],
    mesh=_mesh,
    scratch_types=[
        pltpu.VMEM((_N,), _i32),
        pltpu.VMEM((64, _HID), _f32),
        pltpu.VMEM((8, _HID), _f32),
        pltpu.VMEM((8, _HID), _f32),
        pltpu.VMEM((8, 16), _f32),
        pltpu.SemaphoreType.DMA,
    ],
    compiler_params=pltpu.CompilerParams(use_tc_tiling_on_sc=False, needs_layout_passes=False),
)
def _k_readout(h_hbm, batch_hbm, mean_out, max_out,
               bbuf, chunk, msum, mmax, cnt, sem):
    c = lax.axis_index("c")
    s = lax.axis_index("s")
    w = s * 2 + c
    lo = w * 8
    zero16 = jnp.zeros((16,), _f32)
    lov = jnp.zeros((16,), _i32) + lo
    hiv = lov + 8

    def _per_graph(g, carry):
        pltpu.sync_copy(batch_hbm.at[g], bbuf)

        def _cnt(i, cr):
            n0, n1 = cr
            bv = bbuf[pl.ds(i * 16, 16)]
            n0 = n0 + plsc.all_reduce_population_count(bv < lov)
            n1 = n1 + plsc.all_reduce_population_count(bv < hiv)
            return (n0, n1)

        n0v, n1v = lax.fori_loop(
            0, _N // 16, _cnt,
            (jnp.zeros((16,), _i32), jnp.zeros((16,), _i32)))
        n0 = jnp.max(n0v)
        nn = jnp.max(n1v) - n0

        def _init(r, cr):
            for q in range(16):
                msum[r, pl.ds(q * 16, 16)] = zero16
                mmax[r, pl.ds(q * 16, 16)] = zero16 + _NEG
            cnt[r, :] = zero16
            return cr

        lax.fori_loop(0, 8, _init, 0)

        nch = (nn + 63) // 64

        def _chunkb(k, cr):
            start = n0 + k * 64
            r0 = jnp.minimum(start, _N - 64)
            pltpu.sync_copy(h_hbm.at[g, pl.ds(r0, 64), :], chunk)
            startv = jnp.zeros((16,), _i32) + start

            def _row(r, cr2):
                node = r0 + r
                nodev = jnp.zeros((16,), _i32) + node
                bn16 = plsc.load_gather(bbuf, [nodev])
                valid = (nodev >= startv) & (nodev < (jnp.zeros((16,), _i32) + n0 + nn))
                local = jnp.clip(bn16 - lov, 0, 7)
                lsc = jnp.max(local)
                for q in range(16):
                    v = chunk[r, pl.ds(q * 16, 16)]
                    msum[lsc, pl.ds(q * 16, 16)] = (
                        msum[lsc, pl.ds(q * 16, 16)] + jnp.where(valid, v, 0.0))
                    mmax[lsc, pl.ds(q * 16, 16)] = jnp.maximum(
                        mmax[lsc, pl.ds(q * 16, 16)], jnp.where(valid, v, _NEG))
                cnt[lsc, :] = cnt[lsc, :] + jnp.where(valid, 1.0, 0.0)
                return cr2

            lax.fori_loop(0, 64, _row, 0)
            return cr

        lax.fori_loop(0, nch, _chunkb, 0)

        def _fin(r, cr):
            cv = cnt[r, :]
            denom = jnp.maximum(cv, 1.0)
            empty = cv <= 0.0
            for q in range(16):
                msum[r, pl.ds(q * 16, 16)] = msum[r, pl.ds(q * 16, 16)] / denom
                mx = mmax[r, pl.ds(q * 16, 16)]
                mmax[r, pl.ds(q * 16, 16)] = jnp.where(empty, 0.0, mx)
            return cr

        lax.fori_loop(0, 8, _fin, 0)
        pltpu.sync_copy(msum, mean_out.at[g, pl.ds(lo, 8), :])
        pltpu.sync_copy(mmax, max_out.at[g, pl.ds(lo, 8), :])
        return carry

    lax.fori_loop(0, _NG, _per_graph, 0)


def _head_body(mean_ref, max_ref, drfp_ref, scm_ref, sch_ref,
               mW1_ref, mb1_ref, mW2_ref, mb2_ref,
               hW1_ref, hb1_ref, hW2_ref, hb2_ref, o_ref):
    def dot(a, b):
        return jnp.dot(a, b, preferred_element_type=_f32)

    acc1 = (dot(mean_ref[3], mW1_ref[0:256])
            + dot(max_ref[3], mW1_ref[256:512])
            + dot(mean_ref[4], mW1_ref[512:768])
            + dot(max_ref[4], mW1_ref[768:1024])
            + dot(scm_ref[...], mW1_ref[1024:1027])
            + mb1_ref[...])
    e_mix = dot(jnp.maximum(acc1, 0.0), mW2_ref[...]) + mb2_ref[...]
    acc = hb1_ref[...] + dot(e_mix, hW1_ref[2560:3072])
    for i in range(5):
        acc = acc + dot(mean_ref[i], hW1_ref[i * 512:i * 512 + 256])
        acc = acc + dot(max_ref[i], hW1_ref[i * 512 + 256:(i + 1) * 512])
    acc = acc + dot(drfp_ref[...], hW1_ref[3072:5120])
    acc = acc + dot(sch_ref[...], hW1_ref[5120:5123])
    h = jnp.maximum(acc, 0.0)
    z = dot(h, hW2_ref[...]) + hb2_ref[...]
    o_ref[...] = 1.0 / (1.0 + jnp.exp(-z))


_k_head = pl.pallas_call(
    _head_body,
    out_shape=jax.ShapeDtypeStruct((_B, 3), _f32),
)


def _att_mat(att):
    eye = jnp.eye(_HEADS, dtype=_f32)
    return (att[:, :, None] * eye[:, None, :]).reshape(_HID, _HEADS)


def kernel(params, sm_x, sm_edge_index, sm_batch, p2_x, p2_edge_index,
           p2_batch, p3_x, p3_edge_index, p3_batch, solvent_a_x,
           solvent_a_edge_index, solvent_a_batch, solvent_b_x,
           solvent_b_edge_index, solvent_b_batch, percent_b, temperature,
           residence_time, drfp):
    pr, ps = params["react"], params["solv"]
    x_all = jnp.stack([sm_x, p2_x, p3_x, solvent_a_x, solvent_b_x])
    eis = [sm_edge_index, p2_edge_index, p3_edge_index,
           solvent_a_edge_index, solvent_b_edge_index]
    src_all = jnp.stack([e[0] for e in eis])
    dst_all = jnp.stack([e[1] for e in eis])
    batch_all = jnp.stack([sm_batch, p2_batch, p3_batch,
                           solvent_a_batch, solvent_b_batch])

    w_in = jnp.stack([pr["in_W"], ps["in_W"]])
    b_in = jnp.stack([pr["in_b"], ps["in_b"]])[:, None, :]
    h = _k_in(x_all, w_in, b_in)

    for l in range(4):
        lr, ls = pr["layers"][l], ps["layers"][l]
        w = jnp.stack([lr["W"], ls["W"]])
        vs = jnp.stack([_att_mat(lr["att_src"]), _att_mat(ls["att_src"])])
        vd = jnp.stack([_att_mat(lr["att_dst"]), _att_mat(ls["att_dst"])])
        bias = jnp.stack([lr["bias"], ls["bias"]])[:, None, :]
        g_st, a_s, a_d, ms, md = _k_prep(h, w, vs, vd)
        num_st, den_st = _k_edge(src_all, dst_all,
                                 a_s.reshape(2, _NG, _N * 4),
                                 a_d.reshape(2, _NG, _N * 4), g_st, ms, md)
        h = _k_finish(num_st, den_st, a_s, a_d, g_st, ms, md, bias, h)

    mean_all, max_all = _k_readout(h, batch_all)

    scm = jnp.stack([percent_b, temperature, residence_time], axis=1)
    sch = jnp.stack([temperature, residence_time, percent_b], axis=1)
    return _k_head(mean_all, max_all, drfp, scm, sch,
                   params["mix_W1"], params["mix_b1"][None, :],
                   params["mix_W2"], params["mix_b2"][None, :],
                   params["head_W1"], params["head_b1"][None, :],
                   params["head_W2"], params["head_b2"][None, :])


# speedup vs baseline: 34.2224x; 1.2501x over previous
import functools

import jax
import jax.numpy as jnp
from jax import lax
from jax.experimental import pallas as pl
from jax.experimental.pallas import tpu as pltpu
from jax.experimental.pallas import tpu_sc as plsc

_N = 10000
_E = 320000
_IN = 128
_HID = 256
_HEADS = 8
_HD = 32
_B = 256
_NG = 5
_BN = 1000
_NBLK = _N // _BN
_C = 128
_NCH = 156
_EPW = _NCH * _C
_NTAIL = (_E - 16 * _EPW) // _C
_RPS = _N // 16
_NEG = -1e30

_f32 = jnp.float32
_i32 = jnp.int32


def _sel(g):
    return jnp.where(g < 3, 0, 1)


def _in_body(x_ref, w_ref, b_ref, o_ref):
    o_ref[0] = jnp.dot(x_ref[0], w_ref[0], preferred_element_type=_f32) + b_ref[0]


_k_in = pl.pallas_call(
    _in_body,
    grid=(_NG, _NBLK),
    in_specs=[
        pl.BlockSpec((1, _BN, _IN), lambda g, i: (g, i, 0)),
        pl.BlockSpec((1, _IN, _HID), lambda g, i: (_sel(g), 0, 0)),
        pl.BlockSpec((1, 1, _HID), lambda g, i: (_sel(g), 0, 0)),
    ],
    out_specs=pl.BlockSpec((1, _BN, _HID), lambda g, i: (g, i, 0)),
    out_shape=jax.ShapeDtypeStruct((_NG, _N, _HID), _f32),
)


def _prep_body(h_ref, w_ref, vs_ref, vd_ref, g_ref, as_ref, ad_ref, ms_ref, md_ref):
    i = pl.program_id(1)
    g = jnp.dot(h_ref[0], w_ref[0], preferred_element_type=_f32)
    for q in range(8):
        g_ref[q, 0] = g[:, q * 32:(q + 1) * 32]
    a_s = jnp.dot(g, vs_ref[0], preferred_element_type=_f32)
    a_d = jnp.dot(g, vd_ref[0], preferred_element_type=_f32)
    as_ref[0, 0] = a_s[:, :4]
    as_ref[1, 0] = a_s[:, 4:]
    ad_ref[0, 0] = a_d[:, :4]
    ad_ref[1, 0] = a_d[:, 4:]
    pad = jnp.full((1, 120), _NEG, _f32)
    ms = jnp.concatenate([jnp.max(a_s, axis=0)[None, :], pad], axis=1)[None]
    md = jnp.concatenate([jnp.max(a_d, axis=0)[None, :], pad], axis=1)[None]

    @pl.when(i == 0)
    def _():
        ms_ref[...] = ms
        md_ref[...] = md

    @pl.when(i > 0)
    def _():
        ms_ref[...] = jnp.maximum(ms_ref[...], ms)
        md_ref[...] = jnp.maximum(md_ref[...], md)


_k_prep = pl.pallas_call(
    _prep_body,
    grid=(_NG, _NBLK),
    in_specs=[
        pl.BlockSpec((1, _BN, _HID), lambda g, i: (g, i, 0)),
        pl.BlockSpec((1, _HID, _HID), lambda g, i: (_sel(g), 0, 0)),
        pl.BlockSpec((1, _HID, _HEADS), lambda g, i: (_sel(g), 0, 0)),
        pl.BlockSpec((1, _HID, _HEADS), lambda g, i: (_sel(g), 0, 0)),
    ],
    out_specs=[
        pl.BlockSpec((8, 1, _BN, 32), lambda g, i: (0, g, i, 0)),
        pl.BlockSpec((2, 1, _BN, 4), lambda g, i: (0, g, i, 0)),
        pl.BlockSpec((2, 1, _BN, 4), lambda g, i: (0, g, i, 0)),
        pl.BlockSpec((1, 1, 128), lambda g, i: (g, 0, 0)),
        pl.BlockSpec((1, 1, 128), lambda g, i: (g, 0, 0)),
    ],
    out_shape=[
        jax.ShapeDtypeStruct((8, _NG, _N, 32), _f32),
        jax.ShapeDtypeStruct((2, _NG, _N, 4), _f32),
        jax.ShapeDtypeStruct((2, _NG, _N, 4), _f32),
        jax.ShapeDtypeStruct((_NG, 1, 128), _f32),
        jax.ShapeDtypeStruct((_NG, 1, 128), _f32),
    ],
    compiler_params=pltpu.CompilerParams(
        dimension_semantics=("arbitrary", "arbitrary")),
)


def _finish_body(num_ref, den_ref, as_ref, ad_ref, g_ref, ms_ref, md_ref,
                 b_ref, h_ref, o_ref):
    m = ms_ref[0, :, :8] + md_ref[0, :, :8]
    m = jnp.where(m >= 0.0, m, 0.2 * m)
    halves = []
    for c in (0, 1):
        a = as_ref[c, 0] + ad_ref[c, 0]
        el = jnp.where(a >= 0.0, a, 0.2 * a)
        ex = jnp.exp(el - m[:, c * 4:c * 4 + 4])
        den = den_ref[c, 0][:, :4] + ex
        gh = jnp.concatenate([g_ref[c * 4 + t, 0] for t in range(4)], axis=1)
        num = jnp.concatenate([num_ref[c * 4 + t, 0] for t in range(4)], axis=1)
        cols = []
        for hh in range(4):
            sl = slice(hh * 32, (hh + 1) * 32)
            numc = num[:, sl] + ex[:, hh:hh + 1] * gh[:, sl]
            cols.append(numc / (den[:, hh:hh + 1] + 1e-16))
        halves.append(jnp.concatenate(cols, axis=1))
    o_ref[0] = jnp.concatenate(halves, axis=1) + b_ref[0] + h_ref[0]


_k_finish = pl.pallas_call(
    _finish_body,
    grid=(_NG, _NBLK),
    in_specs=[
        pl.BlockSpec((8, 1, _BN, 32), lambda g, i: (0, g, i, 0)),
        pl.BlockSpec((2, 1, _BN, 8), lambda g, i: (0, g, i, 0)),
        pl.BlockSpec((2, 1, _BN, 4), lambda g, i: (0, g, i, 0)),
        pl.BlockSpec((2, 1, _BN, 4), lambda g, i: (0, g, i, 0)),
        pl.BlockSpec((8, 1, _BN, 32), lambda g, i: (0, g, i, 0)),
        pl.BlockSpec((1, 1, 128), lambda g, i: (g, 0, 0)),
        pl.BlockSpec((1, 1, 128), lambda g, i: (g, 0, 0)),
        pl.BlockSpec((1, 1, _HID), lambda g, i: (_sel(g), 0, 0)),
        pl.BlockSpec((1, _BN, _HID), lambda g, i: (g, i, 0)),
    ],
    out_specs=pl.BlockSpec((1, _BN, _HID), lambda g, i: (g, i, 0)),
    out_shape=jax.ShapeDtypeStruct((_NG, _N, _HID), _f32),
)


_mesh = plsc.VectorSubcoreMesh(
    core_axis_name="c", subcore_axis_name="s", num_cores=2, num_subcores=16)


@functools.partial(
    pl.kernel,
    out_type=[
        jax.ShapeDtypeStruct((8, _NG, _N, 32), _f32),
        jax.ShapeDtypeStruct((2, _NG, _N, 8), _f32),
    ],
    mesh=_mesh,
    scratch_types=[
        pltpu.VMEM((_N * 4,), _f32),
        pltpu.VMEM((_N * 4,), _f32),
        pltpu.VMEM((2, _C), _i32),
        pltpu.VMEM((2, _C), _i32),
        pltpu.VMEM((2, _C, 32), _f32),
        pltpu.VMEM((2, _C, 8), _f32),
        pltpu.VMEM((125, 32), _f32),
        pltpu.VMEM((128, 8), _f32),
        pltpu.VMEM((16,), _f32),
        pltpu.VMEM((16,), _f32),
        pltpu.VMEM((64,), _f32),
        pltpu.VMEM_SHARED((_N, 32), _f32),
        pltpu.VMEM_SHARED((_N, 8), _f32),
        pltpu.SemaphoreType.DMA,
        pltpu.SemaphoreType.DMA,
        pltpu.SemaphoreType.DMA,
        pltpu.SemaphoreType.DMA,
        pltpu.SemaphoreType.DMA,
        pltpu.SemaphoreType.DMA,
        pltpu.SemaphoreType.DMA,
        pltpu.SemaphoreType.DMA,
    ],
    compiler_params=pltpu.CompilerParams(use_tc_tiling_on_sc=False, needs_layout_passes=False),
)
def _k_edge(src_hbm, dst_hbm, as_hbm, ad_hbm, g_hbm, ms_hbm, md_hbm,
            num_out, den_out, asrc_t, adst_t, src_b, dst_b, rows, exb,
            zb, dzb, mt1, mt2, mloc, num_sp, den_sp,
            sIs0, sId0, sIs1, sId1, sG0, sG1, sSd, sSn):
    sIs = [sIs0, sIs1]
    sId = [sId0, sId1]
    sG = [sG0, sG1]
    c = lax.axis_index("c")
    s = lax.axis_index("s")
    zero16 = jnp.zeros((16,), _f32)
    iota = lax.iota(_i32, 16)

    def _zb(i, carry):
        zb[i // 2, pl.ds((i % 2) * 16, 16)] = zero16
        return carry

    lax.fori_loop(0, 250, _zb, 0)

    def _dzb(j, carry):
        for cc in range(8):
            plsc.store_scatter(
                dzb, [j * 16 + iota, jnp.full((16,), cc, _i32)], zero16)
        return carry

    lax.fori_loop(0, 8, _dzb, 0)

    for g in range(_NG):
        pltpu.sync_copy(as_hbm.at[c, g], asrc_t)
        pltpu.sync_copy(ad_hbm.at[c, g], adst_t)
        pltpu.sync_copy(ms_hbm.at[g, 0, pl.ds(0, 16)], mt1)
        pltpu.sync_copy(md_hbm.at[g, 0, pl.ds(0, 16)], mt2)
        mv = mt1[:] + mt2[:]
        mt2[:] = jnp.where(mv >= 0.0, mv, 0.2 * mv)
        for hh in range(4):
            hidx = jnp.zeros((16,), _i32) + (c * 4 + hh)
            mloc[pl.ds(hh * 16, 16)] = plsc.load_gather(mt2, [hidx])

        for p in range(4):
            for t in range(5):
                r0 = s * _RPS + t * 125
                pltpu.sync_copy(zb, num_sp.at[pl.ds(r0, 125), :])
                if p == 0:
                    pltpu.sync_copy(dzb.at[pl.ds(0, 125), :],
                                    den_sp.at[pl.ds(r0, 125), :])

            def _exz(j, carry):
                for bb in range(2):
                    for cc in range(8):
                        plsc.store_scatter(
                            exb.at[bb],
                            [j * 16 + iota, jnp.full((16,), cc, _i32)], zero16)
                return carry

            lax.fori_loop(0, _C // 16, _exz, 0)
            plsc.subcore_barrier()

            def _group_sc(bb, j):
                src16 = src_b[bb, pl.ds(j * 16, 16)]
                dst16 = dst_b[bb, pl.ds(j * 16, 16)]
                e = (plsc.load_gather(asrc_t, [src16 * 4 + p])
                     + plsc.load_gather(adst_t, [dst16 * 4 + p]))
                e = jnp.where(e >= 0.0, e, 0.2 * e)
                ex = jnp.exp(e - mloc[pl.ds(p * 16, 16)])
                plsc.store_scatter(
                    exb.at[bb], [j * 16 + iota, jnp.full((16,), p, _i32)], ex)

            def _scale_sc(bb, ed):
                exv = plsc.load_gather(
                    exb.at[bb], [jnp.zeros((16,), _i32) + ed,
                                 jnp.full((16,), p, _i32)])
                for q in range(2):
                    rows[bb, ed, pl.ds(q * 16, 16)] = (
                        rows[bb, ed, pl.ds(q * 16, 16)] * exv)

            base0 = s * _EPW
            for bb in range(2):
                pltpu.async_copy(src_hbm.at[g, pl.ds(base0 + bb * _C, _C)],
                                 src_b.at[bb], sIs[bb])
                pltpu.async_copy(dst_hbm.at[g, pl.ds(base0 + bb * _C, _C)],
                                 dst_b.at[bb], sId[bb])

            def _pair(k2, carry):
                baseA = s * _EPW + k2 * 2 * _C
                gds = []
                for bb in range(2):
                    base = baseA + bb * _C
                    pltpu.make_async_copy(
                        src_hbm.at[g, pl.ds(base, _C)],
                        src_b.at[bb], sIs[bb]).wait()
                    pltpu.make_async_copy(
                        dst_hbm.at[g, pl.ds(base, _C)],
                        dst_b.at[bb], sId[bb]).wait()
                    gds.append(pltpu.async_copy(
                        g_hbm.at[c * 4 + p, g].at[src_b.at[bb]],
                        rows.at[bb], sG[bb]))
                sds = []
                for bb in range(2):
                    lax.fori_loop(0, _C // 16,
                                  lambda j, cr, bb=bb: (_group_sc(bb, j), cr)[1], 0)
                    gds[bb].wait()
                    lax.fori_loop(0, _C,
                                  lambda ed, cr, bb=bb: (_scale_sc(bb, ed), cr)[1], 0)
                    sds.append(pltpu.async_copy(
                        exb.at[bb], den_sp.at[dst_b.at[bb]], sSd, add=True))
                    sds.append(pltpu.async_copy(
                        rows.at[bb], num_sp.at[dst_b.at[bb]], sSn, add=True))
                for d in sds:
                    d.wait()

                @pl.when(k2 + 1 < _NCH // 2)
                def _():
                    for bb in range(2):
                        base2 = baseA + (2 + bb) * _C
                        pltpu.async_copy(
                            src_hbm.at[g, pl.ds(base2, _C)],
                            src_b.at[bb], sIs[bb])
                        pltpu.async_copy(
                            dst_hbm.at[g, pl.ds(base2, _C)],
                            dst_b.at[bb], sId[bb])
                return carry

            lax.fori_loop(0, _NCH // 2, _pair, 0)

            @pl.when(s < _NTAIL)
            def _():
                base = 16 * _EPW + s * _C
                pltpu.sync_copy(src_hbm.at[g, pl.ds(base, _C)], src_b.at[0])
                pltpu.sync_copy(dst_hbm.at[g, pl.ds(base, _C)], dst_b.at[0])
                pltpu.async_copy(g_hbm.at[c * 4 + p, g].at[src_b.at[0]],
                                 rows.at[0], sG[0]).wait()
                lax.fori_loop(0, _C // 16,
                              lambda j, cr: (_group_sc(0, j), cr)[1], 0)
                lax.fori_loop(0, _C,
                              lambda ed, cr: (_scale_sc(0, ed), cr)[1], 0)
                pltpu.sync_copy(exb.at[0], den_sp.at[dst_b.at[0]], add=True)
                pltpu.sync_copy(rows.at[0], num_sp.at[dst_b.at[0]], add=True)

            plsc.subcore_barrier()
            r0 = s * _RPS
            pltpu.sync_copy(num_sp.at[pl.ds(r0, _RPS), :],
                            num_out.at[c * 4 + p, g, pl.ds(r0, _RPS), :])
            if p == 3:
                pltpu.sync_copy(den_sp.at[pl.ds(r0, _RPS), :],
                                den_out.at[c, g, pl.ds(r0, _RPS), :])


@functools.partial(
    pl.kernel,
    out_type=[
        jax.ShapeDtypeStruct((_NG, _B, _HID), _f32),
        jax.ShapeDtypeStruct((_NG, _B, _HID), _f32),
    ],
    mesh=_mesh,
    scratch_types=[
        pltpu.VMEM((_N,), _i32),
        pltpu.VMEM((64, _HID), _f32),
        pltpu.VMEM((8, _HID), _f32),
        pltpu.VMEM((8, _HID), _f32),
        pltpu.VMEM((8, 16), _f32),
        pltpu.SemaphoreType.DMA,
    ],
    compiler_params=pltpu.CompilerParams(use_tc_tiling_on_sc=False, needs_layout_passes=False),
)
def _k_readout(h_hbm, batch_hbm, mean_out, max_out,
               bbuf, chunk, msum, mmax, cnt, sem):
    c = lax.axis_index("c")
    s = lax.axis_index("s")
    w = s * 2 + c
    lo = w * 8
    zero16 = jnp.zeros((16,), _f32)
    lov = jnp.zeros((16,), _i32) + lo
    hiv = lov + 8

    def _per_graph(g, carry):
        pltpu.sync_copy(batch_hbm.at[g], bbuf)

        def _cnt(i, cr):
            n0, n1 = cr
            bv = bbuf[pl.ds(i * 16, 16)]
            n0 = n0 + plsc.all_reduce_population_count(bv < lov)
            n1 = n1 + plsc.all_reduce_population_count(bv < hiv)
            return (n0, n1)

        n0v, n1v = lax.fori_loop(
            0, _N // 16, _cnt,
            (jnp.zeros((16,), _i32), jnp.zeros((16,), _i32)))
        n0 = jnp.max(n0v)
        nn = jnp.max(n1v) - n0

        def _init(r, cr):
            for q in range(16):
                msum[r, pl.ds(q * 16, 16)] = zero16
                mmax[r, pl.ds(q * 16, 16)] = zero16 + _NEG
            cnt[r, :] = zero16
            return cr

        lax.fori_loop(0, 8, _init, 0)

        nch = (nn + 63) // 64

        def _chunkb(k, cr):
            start = n0 + k * 64
            r0 = jnp.minimum(start, _N - 64)
            pltpu.sync_copy(h_hbm.at[g, pl.ds(r0, 64), :], chunk)
            startv = jnp.zeros((16,), _i32) + start

            def _row(r, cr2):
                node = r0 + r
                nodev = jnp.zeros((16,), _i32) + node
                bn16 = plsc.load_gather(bbuf, [nodev])
                valid = (nodev >= startv) & (nodev < (jnp.zeros((16,), _i32) + n0 + nn))
                local = jnp.clip(bn16 - lov, 0, 7)
                lsc = jnp.max(local)
                for q in range(16):
                    v = chunk[r, pl.ds(q * 16, 16)]
                    msum[lsc, pl.ds(q * 16, 16)] = (
                        msum[lsc, pl.ds(q * 16, 16)] + jnp.where(valid, v, 0.0))
                    mmax[lsc, pl.ds(q * 16, 16)] = jnp.maximum(
                        mmax[lsc, pl.ds(q * 16, 16)], jnp.where(valid, v, _NEG))
                cnt[lsc, :] = cnt[lsc, :] + jnp.where(valid, 1.0, 0.0)
                return cr2

            lax.fori_loop(0, 64, _row, 0)
            return cr

        lax.fori_loop(0, nch, _chunkb, 0)

        def _fin(r, cr):
            cv = cnt[r, :]
            denom = jnp.maximum(cv, 1.0)
            empty = cv <= 0.0
            for q in range(16):
                msum[r, pl.ds(q * 16, 16)] = msum[r, pl.ds(q * 16, 16)] / denom
                mx = mmax[r, pl.ds(q * 16, 16)]
                mmax[r, pl.ds(q * 16, 16)] = jnp.where(empty, 0.0, mx)
            return cr

        lax.fori_loop(0, 8, _fin, 0)
        pltpu.sync_copy(msum, mean_out.at[g, pl.ds(lo, 8), :])
        pltpu.sync_copy(mmax, max_out.at[g, pl.ds(lo, 8), :])
        return carry

    lax.fori_loop(0, _NG, _per_graph, 0)


def _head_body(mean_ref, max_ref, drfp_ref, scm_ref, sch_ref,
               mW1_ref, mb1_ref, mW2_ref, mb2_ref,
               hW1_ref, hb1_ref, hW2_ref, hb2_ref, o_ref):
    def dot(a, b):
        return jnp.dot(a, b, preferred_element_type=_f32)

    acc1 = (dot(mean_ref[3], mW1_ref[0:256])
            + dot(max_ref[3], mW1_ref[256:512])
            + dot(mean_ref[4], mW1_ref[512:768])
            + dot(max_ref[4], mW1_ref[768:1024])
            + dot(scm_ref[...], mW1_ref[1024:1027])
            + mb1_ref[...])
    e_mix = dot(jnp.maximum(acc1, 0.0), mW2_ref[...]) + mb2_ref[...]
    acc = hb1_ref[...] + dot(e_mix, hW1_ref[2560:3072])
    for i in range(5):
        acc = acc + dot(mean_ref[i], hW1_ref[i * 512:i * 512 + 256])
        acc = acc + dot(max_ref[i], hW1_ref[i * 512 + 256:(i + 1) * 512])
    acc = acc + dot(drfp_ref[...], hW1_ref[3072:5120])
    acc = acc + dot(sch_ref[...], hW1_ref[5120:5123])
    h = jnp.maximum(acc, 0.0)
    z = dot(h, hW2_ref[...]) + hb2_ref[...]
    o_ref[...] = 1.0 / (1.0 + jnp.exp(-z))


_k_head = pl.pallas_call(
    _head_body,
    out_shape=jax.ShapeDtypeStruct((_B, 3), _f32),
)


def _att_mat(att):
    eye = jnp.eye(_HEADS, dtype=_f32)
    return (att[:, :, None] * eye[:, None, :]).reshape(_HID, _HEADS)


def kernel(params, sm_x, sm_edge_index, sm_batch, p2_x, p2_edge_index,
           p2_batch, p3_x, p3_edge_index, p3_batch, solvent_a_x,
           solvent_a_edge_index, solvent_a_batch, solvent_b_x,
           solvent_b_edge_index, solvent_b_batch, percent_b, temperature,
           residence_time, drfp):
    pr, ps = params["react"], params["solv"]
    x_all = jnp.stack([sm_x, p2_x, p3_x, solvent_a_x, solvent_b_x])
    eis = [sm_edge_index, p2_edge_index, p3_edge_index,
           solvent_a_edge_index, solvent_b_edge_index]
    src_all = jnp.stack([e[0] for e in eis])
    dst_all = jnp.stack([e[1] for e in eis])
    batch_all = jnp.stack([sm_batch, p2_batch, p3_batch,
                           solvent_a_batch, solvent_b_batch])

    w_in = jnp.stack([pr["in_W"], ps["in_W"]])
    b_in = jnp.stack([pr["in_b"], ps["in_b"]])[:, None, :]
    h = _k_in(x_all, w_in, b_in)

    for l in range(4):
        lr, ls = pr["layers"][l], ps["layers"][l]
        w = jnp.stack([lr["W"], ls["W"]])
        vs = jnp.stack([_att_mat(lr["att_src"]), _att_mat(ls["att_src"])])
        vd = jnp.stack([_att_mat(lr["att_dst"]), _att_mat(ls["att_dst"])])
        bias = jnp.stack([lr["bias"], ls["bias"]])[:, None, :]
        g_st, a_s, a_d, ms, md = _k_prep(h, w, vs, vd)
        num_st, den_st = _k_edge(src_all, dst_all,
                                 a_s.reshape(2, _NG, _N * 4),
                                 a_d.reshape(2, _NG, _N * 4), g_st, ms, md)
        h = _k_finish(num_st, den_st, a_s, a_d, g_st, ms, md, bias, h)

    mean_all, max_all = _k_readout(h, batch_all)

    scm = jnp.stack([percent_b, temperature, residence_time], axis=1)
    sch = jnp.stack([temperature, residence_time, percent_b], axis=1)
    return _k_head(mean_all, max_all, drfp, scm, sch,
                   params["mix_W1"], params["mix_b1"][None, :],
                   params["mix_W2"], params["mix_b2"][None, :],
                   params["head_W1"], params["head_b1"][None, :],
                   params["head_W2"], params["head_b2"][None, :])


# speedup vs baseline: 34.9870x; 1.0223x over previous
import functools

import jax
import jax.numpy as jnp
from jax import lax
from jax.experimental import pallas as pl
from jax.experimental.pallas import tpu as pltpu
from jax.experimental.pallas import tpu_sc as plsc

_N = 10000
_E = 320000
_IN = 128
_HID = 256
_HEADS = 8
_HD = 32
_B = 256
_NG = 5
_BN = 1000
_NBLK = _N // _BN
_C = 128
_NCH = 156
_EPW = _NCH * _C
_NTAIL = (_E - 16 * _EPW) // _C
_RPS = _N // 16
_NEG = -1e30

_f32 = jnp.float32
_i32 = jnp.int32


def _sel(g):
    return jnp.where(g < 3, 0, 1)


def _in_body(x_ref, w_ref, b_ref, o_ref):
    o_ref[0] = jnp.dot(x_ref[0], w_ref[0], preferred_element_type=_f32) + b_ref[0]


_k_in = pl.pallas_call(
    _in_body,
    grid=(_NG, _NBLK),
    in_specs=[
        pl.BlockSpec((1, _BN, _IN), lambda g, i: (g, i, 0)),
        pl.BlockSpec((1, _IN, _HID), lambda g, i: (_sel(g), 0, 0)),
        pl.BlockSpec((1, 1, _HID), lambda g, i: (_sel(g), 0, 0)),
    ],
    out_specs=pl.BlockSpec((1, _BN, _HID), lambda g, i: (g, i, 0)),
    out_shape=jax.ShapeDtypeStruct((_NG, _N, _HID), _f32),
)


def _prep_body(h_ref, w_ref, vs_ref, vd_ref, g_ref, as_ref, ad_ref, ms_ref, md_ref):
    i = pl.program_id(1)
    g = jnp.dot(h_ref[0], w_ref[0], preferred_element_type=_f32)
    for q in range(8):
        g_ref[q, 0] = g[:, q * 32:(q + 1) * 32]
    a_s = jnp.dot(g, vs_ref[0], preferred_element_type=_f32)
    a_d = jnp.dot(g, vd_ref[0], preferred_element_type=_f32)
    as_ref[0, 0] = a_s[:, :4]
    as_ref[1, 0] = a_s[:, 4:]
    ad_ref[0, 0] = a_d[:, :4]
    ad_ref[1, 0] = a_d[:, 4:]
    pad = jnp.full((1, 120), _NEG, _f32)
    ms = jnp.concatenate([jnp.max(a_s, axis=0)[None, :], pad], axis=1)[None]
    md = jnp.concatenate([jnp.max(a_d, axis=0)[None, :], pad], axis=1)[None]

    @pl.when(i == 0)
    def _():
        ms_ref[...] = ms
        md_ref[...] = md

    @pl.when(i > 0)
    def _():
        ms_ref[...] = jnp.maximum(ms_ref[...], ms)
        md_ref[...] = jnp.maximum(md_ref[...], md)


_k_prep = pl.pallas_call(
    _prep_body,
    grid=(_NG, _NBLK),
    in_specs=[
        pl.BlockSpec((1, _BN, _HID), lambda g, i: (g, i, 0)),
        pl.BlockSpec((1, _HID, _HID), lambda g, i: (_sel(g), 0, 0)),
        pl.BlockSpec((1, _HID, _HEADS), lambda g, i: (_sel(g), 0, 0)),
        pl.BlockSpec((1, _HID, _HEADS), lambda g, i: (_sel(g), 0, 0)),
    ],
    out_specs=[
        pl.BlockSpec((8, 1, _BN, 32), lambda g, i: (0, g, i, 0)),
        pl.BlockSpec((2, 1, _BN, 4), lambda g, i: (0, g, i, 0)),
        pl.BlockSpec((2, 1, _BN, 4), lambda g, i: (0, g, i, 0)),
        pl.BlockSpec((1, 1, 128), lambda g, i: (g, 0, 0)),
        pl.BlockSpec((1, 1, 128), lambda g, i: (g, 0, 0)),
    ],
    out_shape=[
        jax.ShapeDtypeStruct((8, _NG, _N, 32), _f32),
        jax.ShapeDtypeStruct((2, _NG, _N, 4), _f32),
        jax.ShapeDtypeStruct((2, _NG, _N, 4), _f32),
        jax.ShapeDtypeStruct((_NG, 1, 128), _f32),
        jax.ShapeDtypeStruct((_NG, 1, 128), _f32),
    ],
    compiler_params=pltpu.CompilerParams(
        dimension_semantics=("arbitrary", "arbitrary")),
)


def _finish_body(num_ref, den_ref, as_ref, ad_ref, g_ref, ms_ref, md_ref,
                 b_ref, h_ref, o_ref):
    m = ms_ref[0, :, :8] + md_ref[0, :, :8]
    m = jnp.where(m >= 0.0, m, 0.2 * m)
    halves = []
    for c in (0, 1):
        a = as_ref[c, 0] + ad_ref[c, 0]
        el = jnp.where(a >= 0.0, a, 0.2 * a)
        ex = jnp.exp(el - m[:, c * 4:c * 4 + 4])
        den = den_ref[c, 0][:, :4] + ex
        gh = jnp.concatenate([g_ref[c * 4 + t, 0] for t in range(4)], axis=1)
        num = jnp.concatenate([num_ref[c * 4 + t, 0] for t in range(4)], axis=1)
        cols = []
        for hh in range(4):
            sl = slice(hh * 32, (hh + 1) * 32)
            numc = num[:, sl] + ex[:, hh:hh + 1] * gh[:, sl]
            cols.append(numc / (den[:, hh:hh + 1] + 1e-16))
        halves.append(jnp.concatenate(cols, axis=1))
    o_ref[0] = jnp.concatenate(halves, axis=1) + b_ref[0] + h_ref[0]


_k_finish = pl.pallas_call(
    _finish_body,
    grid=(_NG, _NBLK),
    in_specs=[
        pl.BlockSpec((8, 1, _BN, 32), lambda g, i: (0, g, i, 0)),
        pl.BlockSpec((2, 1, _BN, 8), lambda g, i: (0, g, i, 0)),
        pl.BlockSpec((2, 1, _BN, 4), lambda g, i: (0, g, i, 0)),
        pl.BlockSpec((2, 1, _BN, 4), lambda g, i: (0, g, i, 0)),
        pl.BlockSpec((8, 1, _BN, 32), lambda g, i: (0, g, i, 0)),
        pl.BlockSpec((1, 1, 128), lambda g, i: (g, 0, 0)),
        pl.BlockSpec((1, 1, 128), lambda g, i: (g, 0, 0)),
        pl.BlockSpec((1, 1, _HID), lambda g, i: (_sel(g), 0, 0)),
        pl.BlockSpec((1, _BN, _HID), lambda g, i: (g, i, 0)),
    ],
    out_specs=pl.BlockSpec((1, _BN, _HID), lambda g, i: (g, i, 0)),
    out_shape=jax.ShapeDtypeStruct((_NG, _N, _HID), _f32),
)


_mesh = plsc.VectorSubcoreMesh(
    core_axis_name="c", subcore_axis_name="s", num_cores=2, num_subcores=16)


@functools.partial(
    pl.kernel,
    out_type=[
        jax.ShapeDtypeStruct((8, _NG, _N, 32), _f32),
        jax.ShapeDtypeStruct((2, _NG, _N, 8), _f32),
    ],
    mesh=_mesh,
    scratch_types=[
        pltpu.VMEM((_N * 4,), _f32),
        pltpu.VMEM((_N * 4,), _f32),
        pltpu.VMEM((2, _C), _i32),
        pltpu.VMEM((2, _C), _i32),
        pltpu.VMEM((2, _C), _i32),
        pltpu.VMEM((2, _C, 32), _f32),
        pltpu.VMEM((2, _C, 8), _f32),
        pltpu.VMEM((125, 32), _f32),
        pltpu.VMEM((128, 8), _f32),
        pltpu.VMEM((16,), _f32),
        pltpu.VMEM((16,), _f32),
        pltpu.VMEM((64,), _f32),
        pltpu.VMEM_SHARED((_N, 32), _f32),
        pltpu.VMEM_SHARED((_N, 8), _f32),
        pltpu.SemaphoreType.DMA,
        pltpu.SemaphoreType.DMA,
        pltpu.SemaphoreType.DMA,
        pltpu.SemaphoreType.DMA,
        pltpu.SemaphoreType.DMA,
        pltpu.SemaphoreType.DMA,
        pltpu.SemaphoreType.DMA,
        pltpu.SemaphoreType.DMA,
    ],
    compiler_params=pltpu.CompilerParams(use_tc_tiling_on_sc=False, needs_layout_passes=False),
)
def _k_edge(src_hbm, dst_hbm, as_hbm, ad_hbm, g_hbm, ms_hbm, md_hbm,
            num_out, den_out, asrc_t, adst_t, src_b, dst_b, dstS, rows, exb,
            zb, dzb, mt1, mt2, mloc, num_sp, den_sp,
            sIs0, sId0, sIs1, sId1, sG0, sG1, sSd, sSn):
    sIs = [sIs0, sIs1]
    sId = [sId0, sId1]
    sG = [sG0, sG1]
    c = lax.axis_index("c")
    s = lax.axis_index("s")
    zero16 = jnp.zeros((16,), _f32)
    iota = lax.iota(_i32, 16)

    def _zb(i, carry):
        zb[i // 2, pl.ds((i % 2) * 16, 16)] = zero16
        return carry

    lax.fori_loop(0, 250, _zb, 0)

    def _dzb(j, carry):
        for cc in range(8):
            plsc.store_scatter(
                dzb, [j * 16 + iota, jnp.full((16,), cc, _i32)], zero16)
        return carry

    lax.fori_loop(0, 8, _dzb, 0)

    def _per_graph_edge(g, carry0):
        pltpu.sync_copy(as_hbm.at[c, g], asrc_t)
        pltpu.sync_copy(ad_hbm.at[c, g], adst_t)
        pltpu.sync_copy(ms_hbm.at[g, 0, pl.ds(0, 16)], mt1)
        pltpu.sync_copy(md_hbm.at[g, 0, pl.ds(0, 16)], mt2)
        mv = mt1[:] + mt2[:]
        mt2[:] = jnp.where(mv >= 0.0, mv, 0.2 * mv)
        for hh in range(4):
            hidx = jnp.zeros((16,), _i32) + (c * 4 + hh)
            mloc[pl.ds(hh * 16, 16)] = plsc.load_gather(mt2, [hidx])

        for p in range(4):
            for t in range(5):
                r0 = s * _RPS + t * 125
                pltpu.sync_copy(zb, num_sp.at[pl.ds(r0, 125), :])
                if p == 0:
                    pltpu.sync_copy(dzb.at[pl.ds(0, 125), :],
                                    den_sp.at[pl.ds(r0, 125), :])

            def _exz(j, carry):
                for bb in range(2):
                    for cc in range(8):
                        plsc.store_scatter(
                            exb.at[bb],
                            [j * 16 + iota, jnp.full((16,), cc, _i32)], zero16)
                return carry

            lax.fori_loop(0, _C // 16, _exz, 0)
            plsc.subcore_barrier()

            def _group_sc(bb, j):
                src16 = src_b[bb, pl.ds(j * 16, 16)]
                dst16 = dst_b[bb, pl.ds(j * 16, 16)]
                e = (plsc.load_gather(asrc_t, [src16 * 4 + p])
                     + plsc.load_gather(adst_t, [dst16 * 4 + p]))
                e = jnp.where(e >= 0.0, e, 0.2 * e)
                ex = jnp.exp(e - mloc[pl.ds(p * 16, 16)])
                plsc.store_scatter(
                    exb.at[bb], [j * 16 + iota, jnp.full((16,), p, _i32)], ex)

            def _scale_sc(bb, ed4):
                for u in range(4):
                    ed = ed4 * 4 + u
                    exv = plsc.load_gather(
                        exb.at[bb], [jnp.zeros((16,), _i32) + ed,
                                     jnp.full((16,), p, _i32)])
                    for q in range(2):
                        rows[bb, ed, pl.ds(q * 16, 16)] = (
                            rows[bb, ed, pl.ds(q * 16, 16)] * exv)

            base0 = s * _EPW
            for bb in range(2):
                pltpu.async_copy(src_hbm.at[g, pl.ds(base0 + bb * _C, _C)],
                                 src_b.at[bb], sIs[bb])
                pltpu.async_copy(dst_hbm.at[g, pl.ds(base0 + bb * _C, _C)],
                                 dst_b.at[bb], sId[bb])

            def _pair(k2, carry):
                baseA = s * _EPW + k2 * 2 * _C
                gds = []
                for bb in range(2):
                    base = baseA + bb * _C
                    pltpu.make_async_copy(
                        src_hbm.at[g, pl.ds(base, _C)],
                        src_b.at[bb], sIs[bb]).wait()
                    pltpu.make_async_copy(
                        dst_hbm.at[g, pl.ds(base, _C)],
                        dst_b.at[bb], sId[bb]).wait()
                    gds.append(pltpu.async_copy(
                        g_hbm.at[c * 4 + p, g].at[src_b.at[bb]],
                        rows.at[bb], sG[bb]))
                sds = []
                for bb in range(2):
                    lax.fori_loop(0, _C // 16,
                                  lambda j, cr, bb=bb: (_group_sc(bb, j), cr)[1], 0)
                    gds[bb].wait()
                    lax.fori_loop(0, _C // 4,
                                  lambda e4, cr, bb=bb: (_scale_sc(bb, e4), cr)[1], 0)
                    sds.append(pltpu.async_copy(
                        exb.at[bb], den_sp.at[dst_b.at[bb]], sSd, add=True))
                    sds.append(pltpu.async_copy(
                        rows.at[bb], num_sp.at[dst_b.at[bb]], sSn, add=True))
                for d in sds:
                    d.wait()

                @pl.when(k2 + 1 < _NCH // 2)
                def _():
                    for bb in range(2):
                        base2 = baseA + (2 + bb) * _C
                        pltpu.async_copy(
                            src_hbm.at[g, pl.ds(base2, _C)],
                            src_b.at[bb], sIs[bb])
                        pltpu.async_copy(
                            dst_hbm.at[g, pl.ds(base2, _C)],
                            dst_b.at[bb], sId[bb])
                return carry

            lax.fori_loop(0, _NCH // 2, _pair, 0)

            @pl.when(s < _NTAIL)
            def _():
                base = 16 * _EPW + s * _C
                pltpu.sync_copy(src_hbm.at[g, pl.ds(base, _C)], src_b.at[0])
                pltpu.sync_copy(dst_hbm.at[g, pl.ds(base, _C)], dst_b.at[0])
                pltpu.async_copy(g_hbm.at[c * 4 + p, g].at[src_b.at[0]],
                                 rows.at[0], sG[0]).wait()
                lax.fori_loop(0, _C // 16,
                              lambda j, cr: (_group_sc(0, j), cr)[1], 0)
                lax.fori_loop(0, _C // 4,
                              lambda e4, cr: (_scale_sc(0, e4), cr)[1], 0)
                pltpu.sync_copy(exb.at[0], den_sp.at[dst_b.at[0]], add=True)
                pltpu.sync_copy(rows.at[0], num_sp.at[dst_b.at[0]], add=True)

            plsc.subcore_barrier()
            r0 = s * _RPS
            pltpu.sync_copy(num_sp.at[pl.ds(r0, _RPS), :],
                            num_out.at[c * 4 + p, g, pl.ds(r0, _RPS), :])
            if p == 3:
                pltpu.sync_copy(den_sp.at[pl.ds(r0, _RPS), :],
                                den_out.at[c, g, pl.ds(r0, _RPS), :])
        return carry0

    lax.fori_loop(0, _NG, _per_graph_edge, 0)


@functools.partial(
    pl.kernel,
    out_type=[
        jax.ShapeDtypeStruct((_NG, _B, _HID), _f32),
        jax.ShapeDtypeStruct((_NG, _B, _HID), _f32),
    ],
    mesh=_mesh,
    scratch_types=[
        pltpu.VMEM((_N,), _i32),
        pltpu.VMEM((64, _HID), _f32),
        pltpu.VMEM((8, _HID), _f32),
        pltpu.VMEM((8, _HID), _f32),
        pltpu.VMEM((8, 16), _f32),
        pltpu.SemaphoreType.DMA,
    ],
    compiler_params=pltpu.CompilerParams(use_tc_tiling_on_sc=False, needs_layout_passes=False),
)
def _k_readout(h_hbm, batch_hbm, mean_out, max_out,
               bbuf, chunk, msum, mmax, cnt, sem):
    c = lax.axis_index("c")
    s = lax.axis_index("s")
    w = s * 2 + c
    lo = w * 8
    zero16 = jnp.zeros((16,), _f32)
    lov = jnp.zeros((16,), _i32) + lo
    hiv = lov + 8

    def _per_graph(g, carry):
        pltpu.sync_copy(batch_hbm.at[g], bbuf)

        def _cnt(i, cr):
            n0, n1 = cr
            bv = bbuf[pl.ds(i * 16, 16)]
            n0 = n0 + plsc.all_reduce_population_count(bv < lov)
            n1 = n1 + plsc.all_reduce_population_count(bv < hiv)
            return (n0, n1)

        n0v, n1v = lax.fori_loop(
            0, _N // 16, _cnt,
            (jnp.zeros((16,), _i32), jnp.zeros((16,), _i32)))
        n0 = jnp.max(n0v)
        nn = jnp.max(n1v) - n0

        def _init(r, cr):
            for q in range(16):
                msum[r, pl.ds(q * 16, 16)] = zero16
                mmax[r, pl.ds(q * 16, 16)] = zero16 + _NEG
            cnt[r, :] = zero16
            return cr

        lax.fori_loop(0, 8, _init, 0)

        nch = (nn + 63) // 64

        def _chunkb(k, cr):
            start = n0 + k * 64
            r0 = jnp.minimum(start, _N - 64)
            pltpu.sync_copy(h_hbm.at[g, pl.ds(r0, 64), :], chunk)
            startv = jnp.zeros((16,), _i32) + start

            def _row(r, cr2):
                node = r0 + r
                nodev = jnp.zeros((16,), _i32) + node
                bn16 = plsc.load_gather(bbuf, [nodev])
                valid = (nodev >= startv) & (nodev < (jnp.zeros((16,), _i32) + n0 + nn))
                local = jnp.clip(bn16 - lov, 0, 7)
                lsc = jnp.max(local)
                for q in range(16):
                    v = chunk[r, pl.ds(q * 16, 16)]
                    msum[lsc, pl.ds(q * 16, 16)] = (
                        msum[lsc, pl.ds(q * 16, 16)] + jnp.where(valid, v, 0.0))
                    mmax[lsc, pl.ds(q * 16, 16)] = jnp.maximum(
                        mmax[lsc, pl.ds(q * 16, 16)], jnp.where(valid, v, _NEG))
                cnt[lsc, :] = cnt[lsc, :] + jnp.where(valid, 1.0, 0.0)
                return cr2

            lax.fori_loop(0, 64, _row, 0)
            return cr

        lax.fori_loop(0, nch, _chunkb, 0)

        def _fin(r, cr):
            cv = cnt[r, :]
            denom = jnp.maximum(cv, 1.0)
            empty = cv <= 0.0
            for q in range(16):
                msum[r, pl.ds(q * 16, 16)] = msum[r, pl.ds(q * 16, 16)] / denom
                mx = mmax[r, pl.ds(q * 16, 16)]
                mmax[r, pl.ds(q * 16, 16)] = jnp.where(empty, 0.0, mx)
            return cr

        lax.fori_loop(0, 8, _fin, 0)
        pltpu.sync_copy(msum, mean_out.at[g, pl.ds(lo, 8), :])
        pltpu.sync_copy(mmax, max_out.at[g, pl.ds(lo, 8), :])
        return carry

    lax.fori_loop(0, _NG, _per_graph, 0)


def _head_body(mean_ref, max_ref, drfp_ref, scm_ref, sch_ref,
               mW1_ref, mb1_ref, mW2_ref, mb2_ref,
               hW1_ref, hb1_ref, hW2_ref, hb2_ref, o_ref):
    def dot(a, b):
        return jnp.dot(a, b, preferred_element_type=_f32)

    acc1 = (dot(mean_ref[3], mW1_ref[0:256])
            + dot(max_ref[3], mW1_ref[256:512])
            + dot(mean_ref[4], mW1_ref[512:768])
            + dot(max_ref[4], mW1_ref[768:1024])
            + dot(scm_ref[...], mW1_ref[1024:1027])
            + mb1_ref[...])
    e_mix = dot(jnp.maximum(acc1, 0.0), mW2_ref[...]) + mb2_ref[...]
    acc = hb1_ref[...] + dot(e_mix, hW1_ref[2560:3072])
    for i in range(5):
        acc = acc + dot(mean_ref[i], hW1_ref[i * 512:i * 512 + 256])
        acc = acc + dot(max_ref[i], hW1_ref[i * 512 + 256:(i + 1) * 512])
    acc = acc + dot(drfp_ref[...], hW1_ref[3072:5120])
    acc = acc + dot(sch_ref[...], hW1_ref[5120:5123])
    h = jnp.maximum(acc, 0.0)
    z = dot(h, hW2_ref[...]) + hb2_ref[...]
    o_ref[...] = 1.0 / (1.0 + jnp.exp(-z))


_k_head = pl.pallas_call(
    _head_body,
    out_shape=jax.ShapeDtypeStruct((_B, 3), _f32),
)


def _att_mat(att):
    eye = jnp.eye(_HEADS, dtype=_f32)
    return (att[:, :, None] * eye[:, None, :]).reshape(_HID, _HEADS)


def kernel(params, sm_x, sm_edge_index, sm_batch, p2_x, p2_edge_index,
           p2_batch, p3_x, p3_edge_index, p3_batch, solvent_a_x,
           solvent_a_edge_index, solvent_a_batch, solvent_b_x,
           solvent_b_edge_index, solvent_b_batch, percent_b, temperature,
           residence_time, drfp):
    pr, ps = params["react"], params["solv"]
    x_all = jnp.stack([sm_x, p2_x, p3_x, solvent_a_x, solvent_b_x])
    eis = [sm_edge_index, p2_edge_index, p3_edge_index,
           solvent_a_edge_index, solvent_b_edge_index]
    src_all = jnp.stack([e[0] for e in eis])
    dst_all = jnp.stack([e[1] for e in eis])
    batch_all = jnp.stack([sm_batch, p2_batch, p3_batch,
                           solvent_a_batch, solvent_b_batch])

    w_in = jnp.stack([pr["in_W"], ps["in_W"]])
    b_in = jnp.stack([pr["in_b"], ps["in_b"]])[:, None, :]
    h = _k_in(x_all, w_in, b_in)

    for l in range(4):
        lr, ls = pr["layers"][l], ps["layers"][l]
        w = jnp.stack([lr["W"], ls["W"]])
        vs = jnp.stack([_att_mat(lr["att_src"]), _att_mat(ls["att_src"])])
        vd = jnp.stack([_att_mat(lr["att_dst"]), _att_mat(ls["att_dst"])])
        bias = jnp.stack([lr["bias"], ls["bias"]])[:, None, :]
        g_st, a_s, a_d, ms, md = _k_prep(h, w, vs, vd)
        num_st, den_st = _k_edge(src_all, dst_all,
                                 a_s.reshape(2, _NG, _N * 4),
                                 a_d.reshape(2, _NG, _N * 4), g_st, ms, md)
        h = _k_finish(num_st, den_st, a_s, a_d, g_st, ms, md, bias, h)

    mean_all, max_all = _k_readout(h, batch_all)

    scm = jnp.stack([percent_b, temperature, residence_time], axis=1)
    sch = jnp.stack([temperature, residence_time, percent_b], axis=1)
    return _k_head(mean_all, max_all, drfp, scm, sch,
                   params["mix_W1"], params["mix_b1"][None, :],
                   params["mix_W2"], params["mix_b2"][None, :],
                   params["head_W1"], params["head_b1"][None, :],
                   params["head_W2"], params["head_b2"][None, :])


# speedup vs baseline: 37.4187x; 1.0695x over previous
import functools

import jax
import jax.numpy as jnp
from jax import lax
from jax.experimental import pallas as pl
from jax.experimental.pallas import tpu as pltpu
from jax.experimental.pallas import tpu_sc as plsc

_N = 10000
_E = 320000
_IN = 128
_HID = 256
_HEADS = 8
_HD = 32
_B = 256
_NG = 5
_BN = 1000
_NBLK = _N // _BN
_C = 128
_NCH = 156
_EPW = _NCH * _C
_NTAIL = (_E - 16 * _EPW) // _C
_RPS = _N // 16
_NEG = -1e30

_f32 = jnp.float32
_i32 = jnp.int32


def _sel(g):
    return jnp.where(g < 3, 0, 1)


def _in_body(x_ref, w_ref, b_ref, o_ref):
    o_ref[0] = jnp.dot(x_ref[0], w_ref[0], preferred_element_type=_f32) + b_ref[0]


_k_in = pl.pallas_call(
    _in_body,
    grid=(_NG, _NBLK),
    in_specs=[
        pl.BlockSpec((1, _BN, _IN), lambda g, i: (g, i, 0)),
        pl.BlockSpec((1, _IN, _HID), lambda g, i: (_sel(g), 0, 0)),
        pl.BlockSpec((1, 1, _HID), lambda g, i: (_sel(g), 0, 0)),
    ],
    out_specs=pl.BlockSpec((1, _BN, _HID), lambda g, i: (g, i, 0)),
    out_shape=jax.ShapeDtypeStruct((_NG, _N, _HID), _f32),
)


def _prep_body(h_ref, w_ref, vs_ref, vd_ref, g_ref, as_ref, ad_ref, ms_ref, md_ref):
    i = pl.program_id(1)
    g = jnp.dot(h_ref[0], w_ref[0], preferred_element_type=_f32)
    for q in range(8):
        g_ref[q, 0] = g[:, q * 32:(q + 1) * 32]
    a_s = jnp.dot(g, vs_ref[0], preferred_element_type=_f32)
    a_d = jnp.dot(g, vd_ref[0], preferred_element_type=_f32)
    as_ref[0, 0] = a_s[:, :4]
    as_ref[1, 0] = a_s[:, 4:]
    ad_ref[0, 0] = a_d[:, :4]
    ad_ref[1, 0] = a_d[:, 4:]
    pad = jnp.full((1, 120), _NEG, _f32)
    ms = jnp.concatenate([jnp.max(a_s, axis=0)[None, :], pad], axis=1)[None]
    md = jnp.concatenate([jnp.max(a_d, axis=0)[None, :], pad], axis=1)[None]

    @pl.when(i == 0)
    def _():
        ms_ref[...] = ms
        md_ref[...] = md

    @pl.when(i > 0)
    def _():
        ms_ref[...] = jnp.maximum(ms_ref[...], ms)
        md_ref[...] = jnp.maximum(md_ref[...], md)


_k_prep = pl.pallas_call(
    _prep_body,
    grid=(_NG, _NBLK),
    in_specs=[
        pl.BlockSpec((1, _BN, _HID), lambda g, i: (g, i, 0)),
        pl.BlockSpec((1, _HID, _HID), lambda g, i: (_sel(g), 0, 0)),
        pl.BlockSpec((1, _HID, _HEADS), lambda g, i: (_sel(g), 0, 0)),
        pl.BlockSpec((1, _HID, _HEADS), lambda g, i: (_sel(g), 0, 0)),
    ],
    out_specs=[
        pl.BlockSpec((8, 1, _BN, 32), lambda g, i: (0, g, i, 0)),
        pl.BlockSpec((2, 1, _BN, 4), lambda g, i: (0, g, i, 0)),
        pl.BlockSpec((2, 1, _BN, 4), lambda g, i: (0, g, i, 0)),
        pl.BlockSpec((1, 1, 128), lambda g, i: (g, 0, 0)),
        pl.BlockSpec((1, 1, 128), lambda g, i: (g, 0, 0)),
    ],
    out_shape=[
        jax.ShapeDtypeStruct((8, _NG, _N, 32), _f32),
        jax.ShapeDtypeStruct((2, _NG, _N, 4), _f32),
        jax.ShapeDtypeStruct((2, _NG, _N, 4), _f32),
        jax.ShapeDtypeStruct((_NG, 1, 128), _f32),
        jax.ShapeDtypeStruct((_NG, 1, 128), _f32),
    ],
    compiler_params=pltpu.CompilerParams(
        dimension_semantics=("arbitrary", "arbitrary")),
)


def _finish_body(num_ref, den_ref, as_ref, ad_ref, g_ref, ms_ref, md_ref,
                 b_ref, h_ref, o_ref):
    m = ms_ref[0, :, :8] + md_ref[0, :, :8]
    m = jnp.where(m >= 0.0, m, 0.2 * m)
    halves = []
    for c in (0, 1):
        a = as_ref[c, 0] + ad_ref[c, 0]
        el = jnp.where(a >= 0.0, a, 0.2 * a)
        ex = jnp.exp(el - m[:, c * 4:c * 4 + 4])
        den = den_ref[c, 0][:, :4] + ex
        gh = jnp.concatenate([g_ref[c * 4 + t, 0] for t in range(4)], axis=1)
        num = jnp.concatenate([num_ref[c * 4 + t, 0] for t in range(4)], axis=1)
        cols = []
        for hh in range(4):
            sl = slice(hh * 32, (hh + 1) * 32)
            numc = num[:, sl] + ex[:, hh:hh + 1] * gh[:, sl]
            cols.append(numc / (den[:, hh:hh + 1] + 1e-16))
        halves.append(jnp.concatenate(cols, axis=1))
    o_ref[0] = jnp.concatenate(halves, axis=1) + b_ref[0] + h_ref[0]


_k_finish = pl.pallas_call(
    _finish_body,
    grid=(_NG, _NBLK),
    in_specs=[
        pl.BlockSpec((8, 1, _BN, 32), lambda g, i: (0, g, i, 0)),
        pl.BlockSpec((2, 1, _BN, 8), lambda g, i: (0, g, i, 0)),
        pl.BlockSpec((2, 1, _BN, 4), lambda g, i: (0, g, i, 0)),
        pl.BlockSpec((2, 1, _BN, 4), lambda g, i: (0, g, i, 0)),
        pl.BlockSpec((8, 1, _BN, 32), lambda g, i: (0, g, i, 0)),
        pl.BlockSpec((1, 1, 128), lambda g, i: (g, 0, 0)),
        pl.BlockSpec((1, 1, 128), lambda g, i: (g, 0, 0)),
        pl.BlockSpec((1, 1, _HID), lambda g, i: (_sel(g), 0, 0)),
        pl.BlockSpec((1, _BN, _HID), lambda g, i: (g, i, 0)),
    ],
    out_specs=pl.BlockSpec((1, _BN, _HID), lambda g, i: (g, i, 0)),
    out_shape=jax.ShapeDtypeStruct((_NG, _N, _HID), _f32),
)


_mesh = plsc.VectorSubcoreMesh(
    core_axis_name="c", subcore_axis_name="s", num_cores=2, num_subcores=16)


@functools.partial(
    pl.kernel,
    out_type=[
        jax.ShapeDtypeStruct((8, _NG, _N, 32), _f32),
        jax.ShapeDtypeStruct((2, _NG, _N, 8), _f32),
    ],
    mesh=_mesh,
    scratch_types=[
        pltpu.VMEM((_N * 4,), _f32),
        pltpu.VMEM((_N * 4,), _f32),
        pltpu.VMEM((2, _C), _i32),
        pltpu.VMEM((2, _C), _i32),
        pltpu.VMEM((2, _C), _i32),
        pltpu.VMEM((2, _C, 32), _f32),
        pltpu.VMEM((2, _C, 8), _f32),
        pltpu.VMEM((125, 32), _f32),
        pltpu.VMEM((128, 8), _f32),
        pltpu.VMEM((16,), _f32),
        pltpu.VMEM((16,), _f32),
        pltpu.VMEM((64,), _f32),
        pltpu.VMEM_SHARED((_N, 32), _f32),
        pltpu.VMEM_SHARED((_N, 8), _f32),
        pltpu.SemaphoreType.DMA,
        pltpu.SemaphoreType.DMA,
        pltpu.SemaphoreType.DMA,
        pltpu.SemaphoreType.DMA,
        pltpu.SemaphoreType.DMA,
        pltpu.SemaphoreType.DMA,
        pltpu.SemaphoreType.DMA,
        pltpu.SemaphoreType.DMA,
    ],
    compiler_params=pltpu.CompilerParams(use_tc_tiling_on_sc=False, needs_layout_passes=False),
)
def _k_edge(src_hbm, dst_hbm, as_hbm, ad_hbm, g_hbm, ms_hbm, md_hbm,
            num_out, den_out, asrc_t, adst_t, src_b, dst_b, dstS, rows, exb,
            zb, dzb, mt1, mt2, mloc, num_sp, den_sp,
            sIs0, sId0, sIs1, sId1, sG0, sG1, sSd, sSn):
    sIs = [sIs0, sIs1]
    sId = [sId0, sId1]
    sG = [sG0, sG1]
    c = lax.axis_index("c")
    s = lax.axis_index("s")
    zero16 = jnp.zeros((16,), _f32)
    iota = lax.iota(_i32, 16)

    def _zb(i, carry):
        zb[i // 2, pl.ds((i % 2) * 16, 16)] = zero16
        return carry

    lax.fori_loop(0, 250, _zb, 0)

    def _dzb(j, carry):
        for cc in range(8):
            plsc.store_scatter(
                dzb, [j * 16 + iota, jnp.full((16,), cc, _i32)], zero16)
        return carry

    lax.fori_loop(0, 8, _dzb, 0)

    def _per_graph_edge(g, carry0):
        pltpu.sync_copy(as_hbm.at[c, g], asrc_t)
        pltpu.sync_copy(ad_hbm.at[c, g], adst_t)
        pltpu.sync_copy(ms_hbm.at[g, 0, pl.ds(0, 16)], mt1)
        pltpu.sync_copy(md_hbm.at[g, 0, pl.ds(0, 16)], mt2)
        mv = mt1[:] + mt2[:]
        mt2[:] = jnp.where(mv >= 0.0, mv, 0.2 * mv)
        for hh in range(4):
            hidx = jnp.zeros((16,), _i32) + (c * 4 + hh)
            mloc[pl.ds(hh * 16, 16)] = plsc.load_gather(mt2, [hidx])

        for p in range(4):
            for t in range(5):
                r0 = s * _RPS + t * 125
                pltpu.sync_copy(zb, num_sp.at[pl.ds(r0, 125), :])
                if p == 0:
                    pltpu.sync_copy(dzb.at[pl.ds(0, 125), :],
                                    den_sp.at[pl.ds(r0, 125), :])

            def _exz(j, carry):
                for bb in range(2):
                    for cc in range(8):
                        plsc.store_scatter(
                            exb.at[bb],
                            [j * 16 + iota, jnp.full((16,), cc, _i32)], zero16)
                return carry

            lax.fori_loop(0, _C // 16, _exz, 0)
            plsc.subcore_barrier()

            def _group_sc(bb, j):
                src16 = src_b[bb, pl.ds(j * 16, 16)]
                dst16 = dst_b[bb, pl.ds(j * 16, 16)]
                e = (plsc.load_gather(asrc_t, [src16 * 4 + p])
                     + plsc.load_gather(adst_t, [dst16 * 4 + p]))
                e = jnp.where(e >= 0.0, e, 0.2 * e)
                ex = jnp.exp(e - mloc[pl.ds(p * 16, 16)])
                plsc.store_scatter(
                    exb.at[bb], [j * 16 + iota, jnp.full((16,), p, _i32)], ex)

            def _scale_sc(bb, ed4):
                for u in range(4):
                    ed = ed4 * 4 + u
                    exv = plsc.load_gather(
                        exb.at[bb], [jnp.zeros((16,), _i32) + ed,
                                     jnp.full((16,), p, _i32)])
                    for q in range(2):
                        rows[bb, ed, pl.ds(q * 16, 16)] = (
                            rows[bb, ed, pl.ds(q * 16, 16)] * exv)

            base0 = s * _EPW
            for bb in range(2):
                pltpu.async_copy(src_hbm.at[g, pl.ds(base0 + bb * _C, _C)],
                                 src_b.at[bb], sIs[bb])
                pltpu.async_copy(dst_hbm.at[g, pl.ds(base0 + bb * _C, _C)],
                                 dst_b.at[bb], sId[bb])

            def _drain_scatters():
                for bb in range(2):
                    pltpu.make_async_copy(
                        exb.at[bb], den_sp.at[dstS.at[bb]], sSd).wait()
                    pltpu.make_async_copy(
                        rows.at[bb], num_sp.at[dstS.at[bb]], sSn).wait()

            def _pair(k2, carry):
                baseA = s * _EPW + k2 * 2 * _C

                @pl.when(k2 > 0)
                def _():
                    _drain_scatters()

                gds = []
                for bb in range(2):
                    base = baseA + bb * _C
                    pltpu.make_async_copy(
                        src_hbm.at[g, pl.ds(base, _C)],
                        src_b.at[bb], sIs[bb]).wait()
                    pltpu.make_async_copy(
                        dst_hbm.at[g, pl.ds(base, _C)],
                        dst_b.at[bb], sId[bb]).wait()
                    gds.append(pltpu.async_copy(
                        g_hbm.at[c * 4 + p, g].at[src_b.at[bb]],
                        rows.at[bb], sG[bb]))
                for bb in range(2):
                    lax.fori_loop(0, _C // 16,
                                  lambda j, cr, bb=bb: (_group_sc(bb, j), cr)[1], 0)

                    def _cpidx(j, cr, bb=bb):
                        dstS[bb, pl.ds(j * 16, 16)] = dst_b[bb, pl.ds(j * 16, 16)]
                        return cr

                    lax.fori_loop(0, _C // 16, _cpidx, 0)
                    gds[bb].wait()
                    lax.fori_loop(0, _C // 4,
                                  lambda e4, cr, bb=bb: (_scale_sc(bb, e4), cr)[1], 0)
                    pltpu.async_copy(
                        exb.at[bb], den_sp.at[dstS.at[bb]], sSd, add=True)
                    pltpu.async_copy(
                        rows.at[bb], num_sp.at[dstS.at[bb]], sSn, add=True)

                @pl.when(k2 + 1 < _NCH // 2)
                def _():
                    for bb in range(2):
                        base2 = baseA + (2 + bb) * _C
                        pltpu.async_copy(
                            src_hbm.at[g, pl.ds(base2, _C)],
                            src_b.at[bb], sIs[bb])
                        pltpu.async_copy(
                            dst_hbm.at[g, pl.ds(base2, _C)],
                            dst_b.at[bb], sId[bb])
                return carry

            lax.fori_loop(0, _NCH // 2, _pair, 0)
            _drain_scatters()

            @pl.when(s < _NTAIL)
            def _():
                base = 16 * _EPW + s * _C
                pltpu.sync_copy(src_hbm.at[g, pl.ds(base, _C)], src_b.at[0])
                pltpu.sync_copy(dst_hbm.at[g, pl.ds(base, _C)], dst_b.at[0])
                pltpu.async_copy(g_hbm.at[c * 4 + p, g].at[src_b.at[0]],
                                 rows.at[0], sG[0]).wait()
                lax.fori_loop(0, _C // 16,
                              lambda j, cr: (_group_sc(0, j), cr)[1], 0)
                lax.fori_loop(0, _C // 4,
                              lambda e4, cr: (_scale_sc(0, e4), cr)[1], 0)
                pltpu.sync_copy(exb.at[0], den_sp.at[dst_b.at[0]], add=True)
                pltpu.sync_copy(rows.at[0], num_sp.at[dst_b.at[0]], add=True)

            plsc.subcore_barrier()
            r0 = s * _RPS
            pltpu.sync_copy(num_sp.at[pl.ds(r0, _RPS), :],
                            num_out.at[c * 4 + p, g, pl.ds(r0, _RPS), :])
            if p == 3:
                pltpu.sync_copy(den_sp.at[pl.ds(r0, _RPS), :],
                                den_out.at[c, g, pl.ds(r0, _RPS), :])
        return carry0

    lax.fori_loop(0, _NG, _per_graph_edge, 0)


@functools.partial(
    pl.kernel,
    out_type=[
        jax.ShapeDtypeStruct((_NG, _B, _HID), _f32),
        jax.ShapeDtypeStruct((_NG, _B, _HID), _f32),
    ],
    mesh=_mesh,
    scratch_types=[
        pltpu.VMEM((_N,), _i32),
        pltpu.VMEM((64, _HID), _f32),
        pltpu.VMEM((8, _HID), _f32),
        pltpu.VMEM((8, _HID), _f32),
        pltpu.VMEM((8, 16), _f32),
        pltpu.SemaphoreType.DMA,
    ],
    compiler_params=pltpu.CompilerParams(use_tc_tiling_on_sc=False, needs_layout_passes=False),
)
def _k_readout(h_hbm, batch_hbm, mean_out, max_out,
               bbuf, chunk, msum, mmax, cnt, sem):
    c = lax.axis_index("c")
    s = lax.axis_index("s")
    w = s * 2 + c
    lo = w * 8
    zero16 = jnp.zeros((16,), _f32)
    lov = jnp.zeros((16,), _i32) + lo
    hiv = lov + 8

    def _per_graph(g, carry):
        pltpu.sync_copy(batch_hbm.at[g], bbuf)

        def _cnt(i, cr):
            n0, n1 = cr
            bv = bbuf[pl.ds(i * 16, 16)]
            n0 = n0 + plsc.all_reduce_population_count(bv < lov)
            n1 = n1 + plsc.all_reduce_population_count(bv < hiv)
            return (n0, n1)

        n0v, n1v = lax.fori_loop(
            0, _N // 16, _cnt,
            (jnp.zeros((16,), _i32), jnp.zeros((16,), _i32)))
        n0 = jnp.max(n0v)
        nn = jnp.max(n1v) - n0

        def _init(r, cr):
            for q in range(16):
                msum[r, pl.ds(q * 16, 16)] = zero16
                mmax[r, pl.ds(q * 16, 16)] = zero16 + _NEG
            cnt[r, :] = zero16
            return cr

        lax.fori_loop(0, 8, _init, 0)

        nch = (nn + 63) // 64

        def _chunkb(k, cr):
            start = n0 + k * 64
            r0 = jnp.minimum(start, _N - 64)
            pltpu.sync_copy(h_hbm.at[g, pl.ds(r0, 64), :], chunk)
            startv = jnp.zeros((16,), _i32) + start

            def _row(r, cr2):
                node = r0 + r
                nodev = jnp.zeros((16,), _i32) + node
                bn16 = plsc.load_gather(bbuf, [nodev])
                valid = (nodev >= startv) & (nodev < (jnp.zeros((16,), _i32) + n0 + nn))
                local = jnp.clip(bn16 - lov, 0, 7)
                lsc = jnp.max(local)
                for q in range(16):
                    v = chunk[r, pl.ds(q * 16, 16)]
                    msum[lsc, pl.ds(q * 16, 16)] = (
                        msum[lsc, pl.ds(q * 16, 16)] + jnp.where(valid, v, 0.0))
                    mmax[lsc, pl.ds(q * 16, 16)] = jnp.maximum(
                        mmax[lsc, pl.ds(q * 16, 16)], jnp.where(valid, v, _NEG))
                cnt[lsc, :] = cnt[lsc, :] + jnp.where(valid, 1.0, 0.0)
                return cr2

            lax.fori_loop(0, 64, _row, 0)
            return cr

        lax.fori_loop(0, nch, _chunkb, 0)

        def _fin(r, cr):
            cv = cnt[r, :]
            denom = jnp.maximum(cv, 1.0)
            empty = cv <= 0.0
            for q in range(16):
                msum[r, pl.ds(q * 16, 16)] = msum[r, pl.ds(q * 16, 16)] / denom
                mx = mmax[r, pl.ds(q * 16, 16)]
                mmax[r, pl.ds(q * 16, 16)] = jnp.where(empty, 0.0, mx)
            return cr

        lax.fori_loop(0, 8, _fin, 0)
        pltpu.sync_copy(msum, mean_out.at[g, pl.ds(lo, 8), :])
        pltpu.sync_copy(mmax, max_out.at[g, pl.ds(lo, 8), :])
        return carry

    lax.fori_loop(0, _NG, _per_graph, 0)


def _head_body(mean_ref, max_ref, drfp_ref, scm_ref, sch_ref,
               mW1_ref, mb1_ref, mW2_ref, mb2_ref,
               hW1_ref, hb1_ref, hW2_ref, hb2_ref, o_ref):
    def dot(a, b):
        return jnp.dot(a, b, preferred_element_type=_f32)

    acc1 = (dot(mean_ref[3], mW1_ref[0:256])
            + dot(max_ref[3], mW1_ref[256:512])
            + dot(mean_ref[4], mW1_ref[512:768])
            + dot(max_ref[4], mW1_ref[768:1024])
            + dot(scm_ref[...], mW1_ref[1024:1027])
            + mb1_ref[...])
    e_mix = dot(jnp.maximum(acc1, 0.0), mW2_ref[...]) + mb2_ref[...]
    acc = hb1_ref[...] + dot(e_mix, hW1_ref[2560:3072])
    for i in range(5):
        acc = acc + dot(mean_ref[i], hW1_ref[i * 512:i * 512 + 256])
        acc = acc + dot(max_ref[i], hW1_ref[i * 512 + 256:(i + 1) * 512])
    acc = acc + dot(drfp_ref[...], hW1_ref[3072:5120])
    acc = acc + dot(sch_ref[...], hW1_ref[5120:5123])
    h = jnp.maximum(acc, 0.0)
    z = dot(h, hW2_ref[...]) + hb2_ref[...]
    o_ref[...] = 1.0 / (1.0 + jnp.exp(-z))


_k_head = pl.pallas_call(
    _head_body,
    out_shape=jax.ShapeDtypeStruct((_B, 3), _f32),
)


def _att_mat(att):
    eye = jnp.eye(_HEADS, dtype=_f32)
    return (att[:, :, None] * eye[:, None, :]).reshape(_HID, _HEADS)


def kernel(params, sm_x, sm_edge_index, sm_batch, p2_x, p2_edge_index,
           p2_batch, p3_x, p3_edge_index, p3_batch, solvent_a_x,
           solvent_a_edge_index, solvent_a_batch, solvent_b_x,
           solvent_b_edge_index, solvent_b_batch, percent_b, temperature,
           residence_time, drfp):
    pr, ps = params["react"], params["solv"]
    x_all = jnp.stack([sm_x, p2_x, p3_x, solvent_a_x, solvent_b_x])
    eis = [sm_edge_index, p2_edge_index, p3_edge_index,
           solvent_a_edge_index, solvent_b_edge_index]
    src_all = jnp.stack([e[0] for e in eis])
    dst_all = jnp.stack([e[1] for e in eis])
    batch_all = jnp.stack([sm_batch, p2_batch, p3_batch,
                           solvent_a_batch, solvent_b_batch])

    w_in = jnp.stack([pr["in_W"], ps["in_W"]])
    b_in = jnp.stack([pr["in_b"], ps["in_b"]])[:, None, :]
    h = _k_in(x_all, w_in, b_in)

    for l in range(4):
        lr, ls = pr["layers"][l], ps["layers"][l]
        w = jnp.stack([lr["W"], ls["W"]])
        vs = jnp.stack([_att_mat(lr["att_src"]), _att_mat(ls["att_src"])])
        vd = jnp.stack([_att_mat(lr["att_dst"]), _att_mat(ls["att_dst"])])
        bias = jnp.stack([lr["bias"], ls["bias"]])[:, None, :]
        g_st, a_s, a_d, ms, md = _k_prep(h, w, vs, vd)
        num_st, den_st = _k_edge(src_all, dst_all,
                                 a_s.reshape(2, _NG, _N * 4),
                                 a_d.reshape(2, _NG, _N * 4), g_st, ms, md)
        h = _k_finish(num_st, den_st, a_s, a_d, g_st, ms, md, bias, h)

    mean_all, max_all = _k_readout(h, batch_all)

    scm = jnp.stack([percent_b, temperature, residence_time], axis=1)
    sch = jnp.stack([temperature, residence_time, percent_b], axis=1)
    return _k_head(mean_all, max_all, drfp, scm, sch,
                   params["mix_W1"], params["mix_b1"][None, :],
                   params["mix_W2"], params["mix_b2"][None, :],
                   params["head_W1"], params["head_b1"][None, :],
                   params["head_W2"], params["head_b2"][None, :])
